# Initial kernel scaffold; baseline (speedup 1.0000x reference)
import jax
import jax.numpy as jnp
from jax.experimental import pallas as pl


def kernel(atomic_number, edge_index, r, atom_emb, Wsg, bsg, Wdg, bdg, Weg, beg, Wsu, bsu, Wdu, bdu, gn, bn, ge, be, Wfc, bfc):
    raise NotImplementedError("write your pallas kernel here")


# speedup vs baseline: 1.4561x; 1.4561x over previous
import functools

import jax
import jax.numpy as jnp
from jax import lax
from jax.experimental import pallas as pl
from jax.experimental.pallas import tpu as pltpu
from jax.experimental.pallas import tpu_sc as plsc

N = 10000
E = 160000
H = 256
L = 3

NP = 10240
EP = 163840
NW = 32
E_PER_TILE = EP // NW
NROWS_PER_TILE = NP // 16
PAD_DST = N + 16
BE = 512
BN = 256

@functools.lru_cache(maxsize=None)
def _make_sc_gather(n_out, n_cols, bg):
    n_per_tile = n_out // NW

    @functools.partial(
        pl.kernel,
        mesh=plsc.VectorSubcoreMesh(core_axis_name="c", subcore_axis_name="s"),
        out_type=jax.ShapeDtypeStruct((n_out, n_cols), jnp.float32),
        scratch_types=[
            pltpu.VMEM((bg,), jnp.int32),
            pltpu.VMEM((bg, n_cols), jnp.float32),
            pltpu.SemaphoreType.DMA,
        ],
    )
    def gather_k(table_hbm, idx_hbm, out_hbm, idx_v, rows_v, sem):
        cid = lax.axis_index("c")
        sid = lax.axis_index("s")
        base = (sid * 2 + cid) * n_per_tile

        def body(b, carry):
            off = base + b * bg
            pltpu.sync_copy(idx_hbm.at[pl.ds(off, bg)], idx_v)
            pltpu.async_copy(table_hbm.at[idx_v], rows_v, sem).wait()
            pltpu.sync_copy(rows_v, out_hbm.at[pl.ds(off, bg)])
            return carry

        lax.fori_loop(0, n_per_tile // bg, body, 0)

    return gather_k


def _gather_e512(table, idx):
    return _make_sc_gather(EP, 512, 128)(table, idx)


def _gather_e256(table, idx):
    return _make_sc_gather(EP, 256, 128)(table, idx)


def _gather_n256(table, idx):
    return _make_sc_gather(NP, 256, 64)(table, idx)


@functools.lru_cache(maxsize=None)
def _make_sc_scatter():
    @functools.partial(
        pl.kernel,
        mesh=plsc.VectorSubcoreMesh(core_axis_name="c", subcore_axis_name="s"),
        out_type=jax.ShapeDtypeStruct((2, 4, NP, 128), jnp.float32),
        scratch_types=[
            pltpu.VMEM((E_PER_TILE // 128, 128), jnp.int32),
            pltpu.VMEM((128, 128), jnp.float32),
            pltpu.VMEM_SHARED((NP, 128), jnp.float32),
        ],
    )
    def scatter_body(dst_hbm, vals_hbm, zeros_hbm, out_hbm, idx_v, vals_v,
                     table_sh):
        cid = lax.axis_index("c")
        sid = lax.axis_index("s")
        base = (sid * 2 + cid) * E_PER_TILE
        row0 = sid * NROWS_PER_TILE
        nblk = E_PER_TILE // 128

        def load_idx(b, carry):
            pltpu.sync_copy(dst_hbm.at[pl.ds(base + b * 128, 128)],
                            idx_v.at[b])
            return carry

        lax.fori_loop(0, nblk, load_idx, 0)

        for c in range(4):
            pltpu.sync_copy(
                zeros_hbm.at[pl.ds(row0, NROWS_PER_TILE)],
                table_sh.at[pl.ds(row0, NROWS_PER_TILE)],
            )
            plsc.subcore_barrier()

            def scat(b, carry):
                pltpu.sync_copy(vals_hbm.at[c, pl.ds(base + b * 128, 128)],
                                vals_v)
                pltpu.sync_copy(vals_v, table_sh.at[idx_v.at[b]], add=True)
                return carry

            lax.fori_loop(0, nblk, scat, 0)
            plsc.subcore_barrier()
            pltpu.sync_copy(
                table_sh.at[pl.ds(row0, NROWS_PER_TILE)],
                out_hbm.at[cid, c, pl.ds(row0, NROWS_PER_TILE)],
            )
            plsc.subcore_barrier()

    return scatter_body


def _scatter_k(dstp, vals4, zeros_n):
    return _make_sc_scatter()(dstp, vals4, zeros_n)


def _xplor(bl):
    r_on, r_cut = 7.5, 8.0
    r2 = bl * bl
    ron2 = r_on * r_on
    rc2 = r_cut * r_cut
    smooth = ((rc2 - r2) ** 2 * (rc2 + 2.0 * r2 - 3.0 * ron2)) / (rc2 - ron2) ** 3
    return jnp.where(bl < r_on, 1.0, jnp.where(bl < r_cut, smooth, 0.0))


def _layernorm(x, g, b, eps=1e-5):
    mu = jnp.mean(x, axis=-1, keepdims=True)
    var = jnp.mean((x - mu) ** 2, axis=-1, keepdims=True)
    return (x - mu) / jnp.sqrt(var + eps) * g + b


def _silu(x):
    return x * jax.nn.sigmoid(x)


def _rbf_body(r_ref, y_ref):
    r = r_ref[...]
    bl = jnp.sqrt(jnp.sum(r * r, axis=1, keepdims=True))
    centers = lax.broadcasted_iota(jnp.int32, (1, H), 1).astype(
        jnp.float32) * (8.0 / (H - 1))
    gamma = 1.0 / (8.0 / (H - 1)) ** 2
    y_ref[...] = jnp.exp(-gamma * (bl - centers) ** 2)


def _rbf(rp):
    return pl.pallas_call(
        _rbf_body,
        grid=(EP // BE,),
        in_specs=[pl.BlockSpec((BE, 8), lambda i: (i, 0))],
        out_specs=pl.BlockSpec((BE, H), lambda i: (i, 0)),
        out_shape=jax.ShapeDtypeStruct((EP, H), jnp.float32),
    )(rp)


def _nodelin_body(x_ref, w_ref, b_ref, o1_ref, o2_ref, o3_ref):
    acc = jnp.dot(x_ref[...], w_ref[...], preferred_element_type=jnp.float32)
    acc = acc + b_ref[...]
    o1_ref[...] = acc[:, :512]
    o2_ref[...] = acc[:, 512:768]
    o3_ref[...] = acc[:, 768:]


def _nodelin(x, wcat, bcat):
    return pl.pallas_call(
        _nodelin_body,
        grid=(NP // BN,),
        in_specs=[
            pl.BlockSpec((BN, H), lambda i: (i, 0)),
            pl.BlockSpec((H, 4 * H), lambda i: (0, 0)),
            pl.BlockSpec((1, 4 * H), lambda i: (0, 0)),
        ],
        out_specs=(
            pl.BlockSpec((BN, 512), lambda i: (i, 0)),
            pl.BlockSpec((BN, 256), lambda i: (i, 0)),
            pl.BlockSpec((BN, 256), lambda i: (i, 0)),
        ),
        out_shape=(
            jax.ShapeDtypeStruct((NP, 512), jnp.float32),
            jax.ShapeDtypeStruct((NP, 256), jnp.float32),
            jax.ShapeDtypeStruct((NP, 256), jnp.float32),
        ),
    )(x, wcat, bcat)


def _edge_body(y_ref, g1_ref, g2_ref, r_ref, weg_ref, beg_ref, ge_ref, be_ref,
               ynew_ref, vals_ref):
    y = y_ref[...]
    g1 = g1_ref[...]
    m = g1[:, :H] + g2_ref[...] + beg_ref[...]
    m = m + jnp.dot(y, weg_ref[...], preferred_element_type=jnp.float32)
    r = r_ref[...]
    bl = jnp.sqrt(jnp.sum(r * r, axis=1, keepdims=True))
    cval = _xplor(bl)
    sig = jax.nn.sigmoid(m) * cval
    u = g1[:, H:] * sig
    vals_ref[0] = sig[:, :128]
    vals_ref[1] = sig[:, 128:]
    vals_ref[2] = u[:, :128]
    vals_ref[3] = u[:, 128:]
    ynew_ref[...] = y + _silu(_layernorm(m, ge_ref[...], be_ref[...]))


def _edge(y, g1, g2, rp, weg, beg, gel, bel):
    return pl.pallas_call(
        _edge_body,
        grid=(EP // BE,),
        in_specs=[
            pl.BlockSpec((BE, H), lambda i: (i, 0)),
            pl.BlockSpec((BE, 512), lambda i: (i, 0)),
            pl.BlockSpec((BE, H), lambda i: (i, 0)),
            pl.BlockSpec((BE, 8), lambda i: (i, 0)),
            pl.BlockSpec((H, H), lambda i: (0, 0)),
            pl.BlockSpec((1, H), lambda i: (0, 0)),
            pl.BlockSpec((1, H), lambda i: (0, 0)),
            pl.BlockSpec((1, H), lambda i: (0, 0)),
        ],
        out_specs=(
            pl.BlockSpec((BE, H), lambda i: (i, 0)),
            pl.BlockSpec((4, BE, 128), lambda i: (0, i, 0)),
        ),
        out_shape=(
            jax.ShapeDtypeStruct((EP, H), jnp.float32),
            jax.ShapeDtypeStruct((4, EP, 128), jnp.float32),
        ),
    )(y, g1, g2, rp, weg, beg, gel, bel)


def _nodeupd_body(x_ref, o3_ref, s_ref, gn_ref, bn_ref, out_ref):
    s = s_ref[...]
    ssum = s[0] + s[1]
    sum_sigma = jnp.concatenate([ssum[0], ssum[1]], axis=1)
    sum_h = jnp.concatenate([ssum[2], ssum[3]], axis=1)
    h = sum_h / (sum_sigma + 1e-6)
    xu = o3_ref[...] + h
    out_ref[...] = x_ref[...] + _silu(_layernorm(xu, gn_ref[...], bn_ref[...]))


def _nodeupd(x, o3, s, gnl, bnl):
    return pl.pallas_call(
        _nodeupd_body,
        grid=(NP // BN,),
        in_specs=[
            pl.BlockSpec((BN, H), lambda i: (i, 0)),
            pl.BlockSpec((BN, H), lambda i: (i, 0)),
            pl.BlockSpec((2, 4, BN, 128), lambda i: (0, 0, i, 0)),
            pl.BlockSpec((1, H), lambda i: (0, 0)),
            pl.BlockSpec((1, H), lambda i: (0, 0)),
        ],
        out_specs=pl.BlockSpec((BN, H), lambda i: (i, 0)),
        out_shape=jax.ShapeDtypeStruct((NP, H), jnp.float32),
    )(x, o3, s, gnl, bnl)


def _readout_body(x_ref, w_ref, out_ref):
    e = jnp.dot(x_ref[...], w_ref[...], preferred_element_type=jnp.float32)
    rows = lax.broadcasted_iota(jnp.int32, (NP, 128), 0)
    e = jnp.where(rows < N, e, 0.0)
    out_ref[...] = jnp.reshape(jnp.sum(e) / N, (1, 1))


def _readout(x, wfc_pad):
    return pl.pallas_call(
        _readout_body,
        in_specs=[
            pl.BlockSpec((NP, H), lambda: (0, 0)),
            pl.BlockSpec((H, 128), lambda: (0, 0)),
        ],
        out_specs=pl.BlockSpec((1, 1), lambda: (0, 0)),
        out_shape=jax.ShapeDtypeStruct((1, 1), jnp.float32),
    )(x, wfc_pad)


def kernel(atomic_number, edge_index, r, atom_emb, Wsg, bsg, Wdg, bdg, Weg,
           beg, Wsu, bsu, Wdu, bdu, gn, bn, ge, be, Wfc, bfc):
    src = edge_index[0].astype(jnp.int32)
    dst = edge_index[1].astype(jnp.int32)
    srcp = jnp.pad(src, (0, EP - E))
    dstp = jnp.pad(dst, (0, EP - E), constant_values=PAD_DST)
    rp = jnp.pad(r, ((0, EP - E), (0, 5)))
    anp = jnp.pad(atomic_number.astype(jnp.int32), (0, NP - N))
    zeros_n = jnp.zeros((NP, 128), jnp.float32)

    x = _gather_n256(atom_emb, anp)
    y = _rbf(rp)

    for l in range(L):
        wcat = jnp.concatenate([Wsg[l], Wsu[l], Wdg[l], Wdu[l]], axis=1)
        bcat = jnp.concatenate([bsg[l], bsu[l], bdg[l], bdu[l]])[None, :]
        o1, o2, o3 = _nodelin(x, wcat, bcat)
        g1 = _gather_e512(o1, srcp)
        g2 = _gather_e256(o2, dstp)
        y, vals4 = _edge(y, g1, g2, rp, Weg[l], beg[l][None, :],
                         ge[l][None, :], be[l][None, :])
        s = _scatter_k(dstp, vals4, zeros_n)
        x = _nodeupd(x, o3, s, gn[l][None, :], bn[l][None, :])

    wfc_pad = jnp.pad(Wfc, ((0, 0), (0, 127)))
    out = _readout(x, wfc_pad)
    return out[0, 0] + bfc[0]


# speedup vs baseline: 1.7057x; 1.1714x over previous
import functools

import jax
import jax.numpy as jnp
from jax import lax
from jax.experimental import pallas as pl
from jax.experimental.pallas import tpu as pltpu
from jax.experimental.pallas import tpu_sc as plsc

N = 10000
E = 160000
H = 256
L = 3

NP = 10240
EP = 163840
NW = 32
E_PER_TILE = EP // NW
NROWS_PER_TILE = NP // 16
PAD_DST = N + 16
BE = 512
BN = 256

def _pipelined_gather(table_hbm, idx_hbm, out_hbm, idx_all, bufs, gsems,
                      wsems, base, npt, bg):
    pltpu.sync_copy(idx_hbm.at[pl.ds(base, npt)], idx_all)
    nblk = npt // bg
    gh = [None] * nblk
    wh = [None] * nblk
    for b in range(nblk + 1):
        if b < nblk:
            if b >= 2:
                wh[b - 2].wait()
            gh[b] = pltpu.async_copy(
                table_hbm.at[idx_all.at[pl.ds(b * bg, bg)]],
                bufs[b % 2], gsems[b % 2])
        if b >= 1:
            gh[b - 1].wait()
            wh[b - 1] = pltpu.async_copy(
                bufs[(b - 1) % 2],
                out_hbm.at[pl.ds(base + (b - 1) * bg, bg)],
                wsems[(b - 1) % 2])
    wh[nblk - 2].wait()
    wh[nblk - 1].wait()


@functools.lru_cache(maxsize=None)
def _make_sc_gather(n_out, n_cols, bg):
    n_per_tile = n_out // NW
    nblk = n_per_tile // bg

    @functools.partial(
        pl.kernel,
        mesh=plsc.VectorSubcoreMesh(core_axis_name="c", subcore_axis_name="s"),
        out_type=jax.ShapeDtypeStruct((n_out, n_cols), jnp.float32),
        scratch_types=[
            pltpu.VMEM((n_per_tile,), jnp.int32),
            pltpu.VMEM((bg, n_cols), jnp.float32),
            pltpu.VMEM((bg, n_cols), jnp.float32),
        ] + [pltpu.SemaphoreType.DMA] * 4,
    )
    def gather_k(t1, i1, o1, idx1, ba, bb, s0, s1, s2, s3):
        cid = lax.axis_index("c")
        sid = lax.axis_index("s")
        wid = sid * 2 + cid
        _pipelined_gather(t1, i1, o1, idx1, [ba, bb], [s0, s1], [s2, s3],
                          wid * n_per_tile, n_per_tile, bg)

    return gather_k


def _gather_edges(table1, src_idx, table2, dst_idx):
    g1 = _make_sc_gather(EP, 512, 64)(table1, src_idx)
    g2 = _make_sc_gather(EP, 256, 128)(table2, dst_idx)
    return g1, g2


def _gather_n256(table, idx):
    return _make_sc_gather(NP, 256, 64)(table, idx)


NBS = E_PER_TILE // 128


@functools.lru_cache(maxsize=None)
def _make_sc_scatter():
    @functools.partial(
        pl.kernel,
        mesh=plsc.VectorSubcoreMesh(core_axis_name="c", subcore_axis_name="s"),
        out_type=jax.ShapeDtypeStruct((2, 4, NP, 128), jnp.float32),
        scratch_types=[
            pltpu.VMEM((NBS, 128), jnp.int32),
            pltpu.VMEM((128, 128), jnp.float32),
            pltpu.VMEM((128, 128), jnp.float32),
            pltpu.VMEM_SHARED((NP, 128), jnp.float32),
        ] + [pltpu.SemaphoreType.DMA] * 5,
    )
    def scatter_body(dst_hbm, vals_hbm, zeros_hbm, out_hbm, idx_v, va, vb,
                     table_sh, l0, l1, a0, a1, isem):
        cid = lax.axis_index("c")
        sid = lax.axis_index("s")
        wid = sid * 2 + cid
        base = wid * E_PER_TILE
        row0 = sid * NROWS_PER_TILE
        ih = [
            pltpu.async_copy(dst_hbm.at[pl.ds(base + b * 128, 128)],
                             idx_v.at[b], isem)
            for b in range(NBS)
        ]
        for h in ih:
            h.wait()
        vbufs = [va, vb]
        lsems = [l0, l1]
        asems = [a0, a1]
        for c in range(4):
            pltpu.sync_copy(zeros_hbm, table_sh.at[pl.ds(row0,
                                                         NROWS_PER_TILE)])
            plsc.subcore_barrier()
            lh = [None] * NBS
            ah = [None] * NBS
            for b in range(NBS + 1):
                if b < NBS:
                    if b >= 2:
                        ah[b - 2].wait()
                    lh[b] = pltpu.async_copy(
                        vals_hbm.at[c, pl.ds(base + b * 128, 128)],
                        vbufs[b % 2], lsems[b % 2])
                if b >= 1:
                    lh[b - 1].wait()
                    ah[b - 1] = pltpu.async_copy(
                        vbufs[(b - 1) % 2], table_sh.at[idx_v.at[b - 1]],
                        asems[(b - 1) % 2], add=True)
            ah[NBS - 2].wait()
            ah[NBS - 1].wait()
            plsc.subcore_barrier()
            pltpu.sync_copy(
                table_sh.at[pl.ds(row0, NROWS_PER_TILE)],
                out_hbm.at[cid, c, pl.ds(row0, NROWS_PER_TILE)],
            )
            plsc.subcore_barrier()

    return scatter_body


def _scatter_k(dstp, vals4, zeros_n):
    return _make_sc_scatter()(dstp, vals4, zeros_n)


def _xplor(bl):
    r_on, r_cut = 7.5, 8.0
    r2 = bl * bl
    ron2 = r_on * r_on
    rc2 = r_cut * r_cut
    smooth = ((rc2 - r2) ** 2 * (rc2 + 2.0 * r2 - 3.0 * ron2)) / (rc2 - ron2) ** 3
    return jnp.where(bl < r_on, 1.0, jnp.where(bl < r_cut, smooth, 0.0))


def _layernorm(x, g, b, eps=1e-5):
    mu = jnp.mean(x, axis=-1, keepdims=True)
    var = jnp.mean((x - mu) ** 2, axis=-1, keepdims=True)
    return (x - mu) / jnp.sqrt(var + eps) * g + b


def _silu(x):
    return x * jax.nn.sigmoid(x)


def _rbf_body(r_ref, y_ref):
    r = r_ref[...]
    bl = jnp.sqrt(jnp.sum(r * r, axis=1, keepdims=True))
    centers = lax.broadcasted_iota(jnp.int32, (1, H), 1).astype(
        jnp.float32) * (8.0 / (H - 1))
    gamma = 1.0 / (8.0 / (H - 1)) ** 2
    y_ref[...] = jnp.exp(-gamma * (bl - centers) ** 2)


def _rbf(rp):
    return pl.pallas_call(
        _rbf_body,
        grid=(EP // BE,),
        in_specs=[pl.BlockSpec((BE, 8), lambda i: (i, 0))],
        out_specs=pl.BlockSpec((BE, H), lambda i: (i, 0)),
        out_shape=jax.ShapeDtypeStruct((EP, H), jnp.float32),
    )(rp)


def _nodelin_body(x_ref, w_ref, b_ref, o1_ref, o2_ref, o3_ref):
    acc = jnp.dot(x_ref[...], w_ref[...], preferred_element_type=jnp.float32)
    acc = acc + b_ref[...]
    o1_ref[...] = acc[:, :512]
    o2_ref[...] = acc[:, 512:768]
    o3_ref[...] = acc[:, 768:]


def _nodelin(x, wcat, bcat):
    return pl.pallas_call(
        _nodelin_body,
        grid=(NP // BN,),
        in_specs=[
            pl.BlockSpec((BN, H), lambda i: (i, 0)),
            pl.BlockSpec((H, 4 * H), lambda i: (0, 0)),
            pl.BlockSpec((1, 4 * H), lambda i: (0, 0)),
        ],
        out_specs=(
            pl.BlockSpec((BN, 512), lambda i: (i, 0)),
            pl.BlockSpec((BN, 256), lambda i: (i, 0)),
            pl.BlockSpec((BN, 256), lambda i: (i, 0)),
        ),
        out_shape=(
            jax.ShapeDtypeStruct((NP, 512), jnp.float32),
            jax.ShapeDtypeStruct((NP, 256), jnp.float32),
            jax.ShapeDtypeStruct((NP, 256), jnp.float32),
        ),
    )(x, wcat, bcat)


def _edge_body(y_ref, g1_ref, g2_ref, r_ref, weg_ref, beg_ref, ge_ref, be_ref,
               ynew_ref, vals_ref):
    y = y_ref[...]
    g1 = g1_ref[...]
    m = g1[:, :H] + g2_ref[...] + beg_ref[...]
    m = m + jnp.dot(y, weg_ref[...], preferred_element_type=jnp.float32)
    r = r_ref[...]
    bl = jnp.sqrt(jnp.sum(r * r, axis=1, keepdims=True))
    cval = _xplor(bl)
    sig = jax.nn.sigmoid(m) * cval
    u = g1[:, H:] * sig
    vals_ref[0] = sig[:, :128]
    vals_ref[1] = sig[:, 128:]
    vals_ref[2] = u[:, :128]
    vals_ref[3] = u[:, 128:]
    ynew_ref[...] = y + _silu(_layernorm(m, ge_ref[...], be_ref[...]))


def _edge(y, g1, g2, rp, weg, beg, gel, bel):
    return pl.pallas_call(
        _edge_body,
        grid=(EP // BE,),
        in_specs=[
            pl.BlockSpec((BE, H), lambda i: (i, 0)),
            pl.BlockSpec((BE, 512), lambda i: (i, 0)),
            pl.BlockSpec((BE, H), lambda i: (i, 0)),
            pl.BlockSpec((BE, 8), lambda i: (i, 0)),
            pl.BlockSpec((H, H), lambda i: (0, 0)),
            pl.BlockSpec((1, H), lambda i: (0, 0)),
            pl.BlockSpec((1, H), lambda i: (0, 0)),
            pl.BlockSpec((1, H), lambda i: (0, 0)),
        ],
        out_specs=(
            pl.BlockSpec((BE, H), lambda i: (i, 0)),
            pl.BlockSpec((4, BE, 128), lambda i: (0, i, 0)),
        ),
        out_shape=(
            jax.ShapeDtypeStruct((EP, H), jnp.float32),
            jax.ShapeDtypeStruct((4, EP, 128), jnp.float32),
        ),
    )(y, g1, g2, rp, weg, beg, gel, bel)


def _nodeupd_body(x_ref, o3_ref, s_ref, gn_ref, bn_ref, out_ref):
    s = s_ref[...]
    ssum = s[0] + s[1]
    sum_sigma = jnp.concatenate([ssum[0], ssum[1]], axis=1)
    sum_h = jnp.concatenate([ssum[2], ssum[3]], axis=1)
    h = sum_h / (sum_sigma + 1e-6)
    xu = o3_ref[...] + h
    out_ref[...] = x_ref[...] + _silu(_layernorm(xu, gn_ref[...], bn_ref[...]))


def _nodeupd(x, o3, s, gnl, bnl):
    return pl.pallas_call(
        _nodeupd_body,
        grid=(NP // BN,),
        in_specs=[
            pl.BlockSpec((BN, H), lambda i: (i, 0)),
            pl.BlockSpec((BN, H), lambda i: (i, 0)),
            pl.BlockSpec((2, 4, BN, 128), lambda i: (0, 0, i, 0)),
            pl.BlockSpec((1, H), lambda i: (0, 0)),
            pl.BlockSpec((1, H), lambda i: (0, 0)),
        ],
        out_specs=pl.BlockSpec((BN, H), lambda i: (i, 0)),
        out_shape=jax.ShapeDtypeStruct((NP, H), jnp.float32),
    )(x, o3, s, gnl, bnl)


def _readout_body(x_ref, w_ref, out_ref):
    e = jnp.dot(x_ref[...], w_ref[...], preferred_element_type=jnp.float32)
    rows = lax.broadcasted_iota(jnp.int32, (NP, 128), 0)
    e = jnp.where(rows < N, e, 0.0)
    out_ref[...] = jnp.reshape(jnp.sum(e) / N, (1, 1))


def _readout(x, wfc_pad):
    return pl.pallas_call(
        _readout_body,
        in_specs=[
            pl.BlockSpec((NP, H), lambda: (0, 0)),
            pl.BlockSpec((H, 128), lambda: (0, 0)),
        ],
        out_specs=pl.BlockSpec((1, 1), lambda: (0, 0)),
        out_shape=jax.ShapeDtypeStruct((1, 1), jnp.float32),
    )(x, wfc_pad)


def kernel(atomic_number, edge_index, r, atom_emb, Wsg, bsg, Wdg, bdg, Weg,
           beg, Wsu, bsu, Wdu, bdu, gn, bn, ge, be, Wfc, bfc):
    src = edge_index[0].astype(jnp.int32)
    dst = edge_index[1].astype(jnp.int32)
    srcp = jnp.pad(src, (0, EP - E))
    dstp = jnp.pad(dst, (0, EP - E), constant_values=PAD_DST)
    rp = jnp.pad(r, ((0, EP - E), (0, 5)))
    anp = jnp.pad(atomic_number.astype(jnp.int32), (0, NP - N))
    zeros_n = jnp.zeros((NROWS_PER_TILE, 128), jnp.float32)

    x = _gather_n256(atom_emb, anp)
    y = _rbf(rp)

    for l in range(L):
        wcat = jnp.concatenate([Wsg[l], Wsu[l], Wdg[l], Wdu[l]], axis=1)
        bcat = jnp.concatenate([bsg[l], bsu[l], bdg[l], bdu[l]])[None, :]
        o1, o2, o3 = _nodelin(x, wcat, bcat)
        g1, g2 = _gather_edges(o1, srcp, o2, dstp)
        y, vals4 = _edge(y, g1, g2, rp, Weg[l], beg[l][None, :],
                         ge[l][None, :], be[l][None, :])
        s = _scatter_k(dstp, vals4, zeros_n)
        x = _nodeupd(x, o3, s, gn[l][None, :], bn[l][None, :])

    wfc_pad = jnp.pad(Wfc, ((0, 0), (0, 127)))
    out = _readout(x, wfc_pad)
    return out[0, 0] + bfc[0]


# speedup vs baseline: 1.8299x; 1.0728x over previous
import functools

import numpy as np

import jax
import jax.numpy as jnp
from jax import lax
from jax.experimental import pallas as pl
from jax.experimental.pallas import tpu as pltpu
from jax.experimental.pallas import tpu_sc as plsc

N = 10000
E = 160000
H = 256
L = 3

NP = 10240
EP = 163840
NW = 32
E_PER_TILE = EP // NW
NROWS_PER_TILE = NP // 16
PAD_DST = N + 16
BE = 512
BN = 256

_SIG = np.concatenate([np.arange(128) * 2, np.arange(128) * 2 + 1])
_SIGINV = np.argsort(_SIG)

def _pipelined_gather(table_hbm, idx_hbm, out_hbm, idx_all, bufs, gsems,
                      wsems, base, npt, bg):
    pltpu.sync_copy(idx_hbm.at[pl.ds(base, npt)], idx_all)
    nblk = npt // bg
    gh = [None] * nblk
    wh = [None] * nblk
    for b in range(nblk + 1):
        if b < nblk:
            if b >= 2:
                wh[b - 2].wait()
            gh[b] = pltpu.async_copy(
                table_hbm.at[idx_all.at[pl.ds(b * bg, bg)]],
                bufs[b % 2], gsems[b % 2])
        if b >= 1:
            gh[b - 1].wait()
            wh[b - 1] = pltpu.async_copy(
                bufs[(b - 1) % 2],
                out_hbm.at[pl.ds(base + (b - 1) * bg, bg)],
                wsems[(b - 1) % 2])
    wh[nblk - 2].wait()
    wh[nblk - 1].wait()


@functools.lru_cache(maxsize=None)
def _make_sc_gather(n_out, n_cols, bg, dtype=jnp.float32):
    n_per_tile = n_out // NW
    nblk = n_per_tile // bg

    @functools.partial(
        pl.kernel,
        mesh=plsc.VectorSubcoreMesh(core_axis_name="c", subcore_axis_name="s"),
        out_type=jax.ShapeDtypeStruct((n_out, n_cols), dtype),
        scratch_types=[
            pltpu.VMEM((n_per_tile,), jnp.int32),
            pltpu.VMEM((bg, n_cols), dtype),
            pltpu.VMEM((bg, n_cols), dtype),
        ] + [pltpu.SemaphoreType.DMA] * 4,
    )
    def gather_k(t1, i1, o1, idx1, ba, bb, s0, s1, s2, s3):
        cid = lax.axis_index("c")
        sid = lax.axis_index("s")
        wid = sid * 2 + cid
        _pipelined_gather(t1, i1, o1, idx1, [ba, bb], [s0, s1], [s2, s3],
                          wid * n_per_tile, n_per_tile, bg)

    return gather_k


def _gather_edges(table1, src_idx, table2, dst_idx):
    t1 = lax.bitcast_convert_type(table1.reshape(NP, 256, 2), jnp.int32)
    t2 = lax.bitcast_convert_type(table2.reshape(NP, 128, 2), jnp.int32)
    g1 = _make_sc_gather(EP, 256, 64, jnp.int32)(t1, src_idx)
    g2 = _make_sc_gather(EP, 128, 128, jnp.int32)(t2, dst_idx)
    return g1, g2


def _gather_n256(table, idx):
    return _make_sc_gather(NP, 256, 64)(table, idx)


NBS = E_PER_TILE // 128


@functools.lru_cache(maxsize=None)
def _make_sc_scatter():
    @functools.partial(
        pl.kernel,
        mesh=plsc.VectorSubcoreMesh(core_axis_name="c", subcore_axis_name="s"),
        out_type=jax.ShapeDtypeStruct((2, 4, NP, 128), jnp.float32),
        scratch_types=[
            pltpu.VMEM((NBS, 128), jnp.int32),
            pltpu.VMEM((128, 128), jnp.float32),
            pltpu.VMEM((128, 128), jnp.float32),
            pltpu.VMEM_SHARED((NP, 128), jnp.float32),
        ] + [pltpu.SemaphoreType.DMA] * 5,
    )
    def scatter_body(dst_hbm, vals_hbm, zeros_hbm, out_hbm, idx_v, va, vb,
                     table_sh, l0, l1, a0, a1, isem):
        cid = lax.axis_index("c")
        sid = lax.axis_index("s")
        wid = sid * 2 + cid
        base = wid * E_PER_TILE
        row0 = sid * NROWS_PER_TILE
        ih = [
            pltpu.async_copy(dst_hbm.at[pl.ds(base + b * 128, 128)],
                             idx_v.at[b], isem)
            for b in range(NBS)
        ]
        for h in ih:
            h.wait()
        vbufs = [va, vb]
        lsems = [l0, l1]
        asems = [a0, a1]
        for c in range(4):
            pltpu.sync_copy(zeros_hbm, table_sh.at[pl.ds(row0,
                                                         NROWS_PER_TILE)])
            plsc.subcore_barrier()
            lh = [None] * NBS
            ah = [None] * NBS
            for b in range(NBS + 1):
                if b < NBS:
                    if b >= 2:
                        ah[b - 2].wait()
                    lh[b] = pltpu.async_copy(
                        vals_hbm.at[c, pl.ds(base + b * 128, 128)],
                        vbufs[b % 2], lsems[b % 2])
                if b >= 1:
                    lh[b - 1].wait()
                    ah[b - 1] = pltpu.async_copy(
                        vbufs[(b - 1) % 2], table_sh.at[idx_v.at[b - 1]],
                        asems[(b - 1) % 2], add=True)
            ah[NBS - 2].wait()
            ah[NBS - 1].wait()
            plsc.subcore_barrier()
            pltpu.sync_copy(
                table_sh.at[pl.ds(row0, NROWS_PER_TILE)],
                out_hbm.at[cid, c, pl.ds(row0, NROWS_PER_TILE)],
            )
            plsc.subcore_barrier()

    return scatter_body


def _scatter_k(dstp, vals4, zeros_n):
    return _make_sc_scatter()(dstp, vals4, zeros_n)


def _xplor(bl):
    r_on, r_cut = 7.5, 8.0
    r2 = bl * bl
    ron2 = r_on * r_on
    rc2 = r_cut * r_cut
    smooth = ((rc2 - r2) ** 2 * (rc2 + 2.0 * r2 - 3.0 * ron2)) / (rc2 - ron2) ** 3
    return jnp.where(bl < r_on, 1.0, jnp.where(bl < r_cut, smooth, 0.0))


def _layernorm(x, g, b, eps=1e-5):
    mu = jnp.mean(x, axis=-1, keepdims=True)
    var = jnp.mean((x - mu) ** 2, axis=-1, keepdims=True)
    return (x - mu) / jnp.sqrt(var + eps) * g + b


def _silu(x):
    return x * jax.nn.sigmoid(x)


def _rbf_body(r_ref, y_ref):
    r = r_ref[...]
    bl = jnp.sqrt(jnp.sum(r * r, axis=1, keepdims=True))
    centers = lax.broadcasted_iota(jnp.int32, (1, H), 1).astype(
        jnp.float32) * (8.0 / (H - 1))
    gamma = 1.0 / (8.0 / (H - 1)) ** 2
    y_ref[...] = jnp.exp(-gamma * (bl - centers) ** 2)


def _rbf(rp):
    return pl.pallas_call(
        _rbf_body,
        grid=(EP // BE,),
        in_specs=[pl.BlockSpec((BE, 8), lambda i: (i, 0))],
        out_specs=pl.BlockSpec((BE, H), lambda i: (i, 0)),
        out_shape=jax.ShapeDtypeStruct((EP, H), jnp.float32),
    )(rp)


def _nodelin_body(x_ref, w_ref, b_ref, o1_ref, o2_ref, o3_ref):
    acc = jnp.dot(x_ref[...], w_ref[...], preferred_element_type=jnp.float32)
    acc = acc + b_ref[...]
    o1_ref[...] = acc[:, :512].astype(jnp.bfloat16)
    o2_ref[...] = acc[:, 512:768].astype(jnp.bfloat16)
    o3_ref[...] = acc[:, 768:]


def _nodelin(x, wcat, bcat):
    return pl.pallas_call(
        _nodelin_body,
        grid=(NP // BN,),
        in_specs=[
            pl.BlockSpec((BN, H), lambda i: (i, 0)),
            pl.BlockSpec((H, 4 * H), lambda i: (0, 0)),
            pl.BlockSpec((1, 4 * H), lambda i: (0, 0)),
        ],
        out_specs=(
            pl.BlockSpec((BN, 512), lambda i: (i, 0)),
            pl.BlockSpec((BN, 256), lambda i: (i, 0)),
            pl.BlockSpec((BN, 256), lambda i: (i, 0)),
        ),
        out_shape=(
            jax.ShapeDtypeStruct((NP, 512), jnp.bfloat16),
            jax.ShapeDtypeStruct((NP, 256), jnp.bfloat16),
            jax.ShapeDtypeStruct((NP, 256), jnp.float32),
        ),
    )(x, wcat, bcat)


def _edge_body(y_ref, g1_ref, g2_ref, r_ref, weg_ref, beg_ref, ge_ref, be_ref,
               ynew_ref, vals_ref):
    y = y_ref[...]
    g1w = g1_ref[...]
    g2w = g2_ref[...]
    mask = jnp.int32(-65536)
    lo1 = lax.bitcast_convert_type(g1w << 16, jnp.float32)
    hi1 = lax.bitcast_convert_type(g1w & mask, jnp.float32)
    lo2 = lax.bitcast_convert_type(g2w << 16, jnp.float32)
    hi2 = lax.bitcast_convert_type(g2w & mask, jnp.float32)
    e_src = jnp.concatenate([lo1[:, :128], hi1[:, :128]], axis=1)
    bh = jnp.concatenate([lo1[:, 128:], hi1[:, 128:]], axis=1)
    e_dst = jnp.concatenate([lo2, hi2], axis=1)
    m = e_src + e_dst + beg_ref[...]
    m = m + jnp.dot(y, weg_ref[...], preferred_element_type=jnp.float32)
    r = r_ref[...]
    bl = jnp.sqrt(jnp.sum(r * r, axis=1, keepdims=True))
    cval = _xplor(bl)
    sig = jax.nn.sigmoid(m) * cval
    u = bh * sig
    vals_ref[0] = sig[:, :128]
    vals_ref[1] = sig[:, 128:]
    vals_ref[2] = u[:, :128]
    vals_ref[3] = u[:, 128:]
    ynew_ref[...] = y + _silu(_layernorm(m, ge_ref[...], be_ref[...]))


def _edge(y, g1, g2, rp, weg, beg, gel, bel):
    return pl.pallas_call(
        _edge_body,
        grid=(EP // BE,),
        in_specs=[
            pl.BlockSpec((BE, H), lambda i: (i, 0)),
            pl.BlockSpec((BE, 256), lambda i: (i, 0)),
            pl.BlockSpec((BE, 128), lambda i: (i, 0)),
            pl.BlockSpec((BE, 8), lambda i: (i, 0)),
            pl.BlockSpec((H, H), lambda i: (0, 0)),
            pl.BlockSpec((1, H), lambda i: (0, 0)),
            pl.BlockSpec((1, H), lambda i: (0, 0)),
            pl.BlockSpec((1, H), lambda i: (0, 0)),
        ],
        out_specs=(
            pl.BlockSpec((BE, H), lambda i: (i, 0)),
            pl.BlockSpec((4, BE, 128), lambda i: (0, i, 0)),
        ),
        out_shape=(
            jax.ShapeDtypeStruct((EP, H), jnp.float32),
            jax.ShapeDtypeStruct((4, EP, 128), jnp.float32),
        ),
    )(y, g1, g2, rp, weg, beg, gel, bel)


def _nodeupd_body(x_ref, o3_ref, s_ref, gn_ref, bn_ref, out_ref):
    s = s_ref[...]
    ssum = s[0] + s[1]
    sum_sigma = jnp.concatenate([ssum[0], ssum[1]], axis=1)
    sum_h = jnp.concatenate([ssum[2], ssum[3]], axis=1)
    h = sum_h / (sum_sigma + 1e-6)
    xu = o3_ref[...] + h
    out_ref[...] = x_ref[...] + _silu(_layernorm(xu, gn_ref[...], bn_ref[...]))


def _nodeupd(x, o3, s, gnl, bnl):
    return pl.pallas_call(
        _nodeupd_body,
        grid=(NP // BN,),
        in_specs=[
            pl.BlockSpec((BN, H), lambda i: (i, 0)),
            pl.BlockSpec((BN, H), lambda i: (i, 0)),
            pl.BlockSpec((2, 4, BN, 128), lambda i: (0, 0, i, 0)),
            pl.BlockSpec((1, H), lambda i: (0, 0)),
            pl.BlockSpec((1, H), lambda i: (0, 0)),
        ],
        out_specs=pl.BlockSpec((BN, H), lambda i: (i, 0)),
        out_shape=jax.ShapeDtypeStruct((NP, H), jnp.float32),
    )(x, o3, s, gnl, bnl)


def _readout_body(x_ref, w_ref, out_ref):
    e = jnp.dot(x_ref[...], w_ref[...], preferred_element_type=jnp.float32)
    rows = lax.broadcasted_iota(jnp.int32, (NP, 128), 0)
    e = jnp.where(rows < N, e, 0.0)
    out_ref[...] = jnp.reshape(jnp.sum(e) / N, (1, 1))


def _readout(x, wfc_pad):
    return pl.pallas_call(
        _readout_body,
        in_specs=[
            pl.BlockSpec((NP, H), lambda: (0, 0)),
            pl.BlockSpec((H, 128), lambda: (0, 0)),
        ],
        out_specs=pl.BlockSpec((1, 1), lambda: (0, 0)),
        out_shape=jax.ShapeDtypeStruct((1, 1), jnp.float32),
    )(x, wfc_pad)


def kernel(atomic_number, edge_index, r, atom_emb, Wsg, bsg, Wdg, bdg, Weg,
           beg, Wsu, bsu, Wdu, bdu, gn, bn, ge, be, Wfc, bfc):
    src = edge_index[0].astype(jnp.int32)
    dst = edge_index[1].astype(jnp.int32)
    srcp = jnp.pad(src, (0, EP - E))
    dstp = jnp.pad(dst, (0, EP - E), constant_values=PAD_DST)
    rp = jnp.pad(r, ((0, EP - E), (0, 5)))
    anp = jnp.pad(atomic_number.astype(jnp.int32), (0, NP - N))
    zeros_n = jnp.zeros((NROWS_PER_TILE, 128), jnp.float32)

    x = _gather_n256(atom_emb, anp)
    y = _rbf(rp)

    for l in range(L):
        wcat = jnp.concatenate([Wsg[l][:, _SIGINV], Wsu[l][:, _SIGINV],
                                Wdg[l][:, _SIGINV], Wdu[l]], axis=1)
        bcat = jnp.concatenate([bsg[l][_SIGINV], bsu[l][_SIGINV],
                                bdg[l][_SIGINV], bdu[l]])[None, :]
        o1, o2, o3 = _nodelin(x, wcat, bcat)
        g1, g2 = _gather_edges(o1, srcp, o2, dstp)
        y, vals4 = _edge(y, g1, g2, rp, Weg[l], beg[l][None, :],
                         ge[l][None, :], be[l][None, :])
        s = _scatter_k(dstp, vals4, zeros_n)
        x = _nodeupd(x, o3, s, gn[l][None, :], bn[l][None, :])

    wfc_pad = jnp.pad(Wfc, ((0, 0), (0, 127)))
    out = _readout(x, wfc_pad)
    return out[0, 0] + bfc[0]


# speedup vs baseline: 1.9823x; 1.0833x over previous
import functools

import numpy as np

import jax
import jax.numpy as jnp
from jax import lax
from jax.experimental import pallas as pl
from jax.experimental.pallas import tpu as pltpu
from jax.experimental.pallas import tpu_sc as plsc

N = 10000
E = 160000
H = 256
L = 3

NP = 10240
EP = 163840
NW = 32
E_PER_TILE = EP // NW
NROWS_PER_TILE = NP // 16
PAD_DST = N + 16
BE = 512
BN = 256

_SIG = np.concatenate([np.arange(128) * 2, np.arange(128) * 2 + 1])
_SIGINV = np.argsort(_SIG)

class _GatherPipe:
    def __init__(self, table_hbm, idx_hbm, out_hbm, idx_all, bufs, gsems,
                 wsems, base, npt, bg):
        pltpu.sync_copy(idx_hbm.at[pl.ds(base, npt)], idx_all)
        self.t, self.o, self.idx = table_hbm, out_hbm, idx_all
        self.bufs, self.gs, self.ws = bufs, gsems, wsems
        self.base, self.bg = base, bg
        self.nblk = npt // bg
        self.gh = [None] * self.nblk
        self.wh = [None] * self.nblk

    def step(self, b):
        if b < self.nblk:
            if b >= 2:
                self.wh[b - 2].wait()
            self.gh[b] = pltpu.async_copy(
                self.t.at[self.idx.at[pl.ds(b * self.bg, self.bg)]],
                self.bufs[b % 2], self.gs[b % 2])
        if 1 <= b <= self.nblk:
            self.gh[b - 1].wait()
            self.wh[b - 1] = pltpu.async_copy(
                self.bufs[(b - 1) % 2],
                self.o.at[pl.ds(self.base + (b - 1) * self.bg, self.bg)],
                self.ws[(b - 1) % 2])

    def drain(self):
        self.wh[self.nblk - 2].wait()
        self.wh[self.nblk - 1].wait()


def _pipelined_gather(table_hbm, idx_hbm, out_hbm, idx_all, bufs, gsems,
                      wsems, base, npt, bg):
    pipe = _GatherPipe(table_hbm, idx_hbm, out_hbm, idx_all, bufs, gsems,
                       wsems, base, npt, bg)
    for b in range(pipe.nblk + 1):
        pipe.step(b)
    pipe.drain()


@functools.lru_cache(maxsize=None)
def _make_sc_gather(n_out, n_cols, bg, dtype=jnp.float32):
    n_per_tile = n_out // NW
    nblk = n_per_tile // bg

    @functools.partial(
        pl.kernel,
        mesh=plsc.VectorSubcoreMesh(core_axis_name="c", subcore_axis_name="s"),
        out_type=jax.ShapeDtypeStruct((n_out, n_cols), dtype),
        scratch_types=[
            pltpu.VMEM((n_per_tile,), jnp.int32),
            pltpu.VMEM((bg, n_cols), dtype),
            pltpu.VMEM((bg, n_cols), dtype),
        ] + [pltpu.SemaphoreType.DMA] * 4,
    )
    def gather_k(t1, i1, o1, idx1, ba, bb, s0, s1, s2, s3):
        cid = lax.axis_index("c")
        sid = lax.axis_index("s")
        wid = sid * 2 + cid
        _pipelined_gather(t1, i1, o1, idx1, [ba, bb], [s0, s1], [s2, s3],
                          wid * n_per_tile, n_per_tile, bg)

    return gather_k


@functools.lru_cache(maxsize=None)
def _make_sc_gather_pair():
    bg = 128
    nblk = E_PER_TILE // bg

    @functools.partial(
        pl.kernel,
        mesh=plsc.VectorSubcoreMesh(core_axis_name="c", subcore_axis_name="s"),
        out_type=(
            jax.ShapeDtypeStruct((EP, 256), jnp.int32),
            jax.ShapeDtypeStruct((EP, 128), jnp.int32),
        ),
        scratch_types=[
            pltpu.VMEM((E_PER_TILE,), jnp.int32),
            pltpu.VMEM((E_PER_TILE,), jnp.int32),
            pltpu.VMEM((bg, 256), jnp.int32),
            pltpu.VMEM((bg, 256), jnp.int32),
            pltpu.VMEM((bg, 128), jnp.int32),
            pltpu.VMEM((bg, 128), jnp.int32),
        ] + [pltpu.SemaphoreType.DMA] * 8,
    )
    def gather_k(t1, i1, t2, i2, o1, o2, idx1, idx2, b1a, b1b, b2a, b2b,
                 s0, s1, s2, s3, s4, s5, s6, s7):
        cid = lax.axis_index("c")
        sid = lax.axis_index("s")
        base = (sid * 2 + cid) * E_PER_TILE
        p1 = _GatherPipe(t1, i1, o1, idx1, [b1a, b1b], [s0, s1], [s2, s3],
                         base, E_PER_TILE, bg)
        p2 = _GatherPipe(t2, i2, o2, idx2, [b2a, b2b], [s4, s5], [s6, s7],
                         base, E_PER_TILE, bg)
        for b in range(nblk + 1):
            p1.step(b)
            p2.step(b)
        p1.drain()
        p2.drain()

    return gather_k


def _gather_edges(table1, src_idx, table2, dst_idx):
    t1 = lax.bitcast_convert_type(table1.reshape(NP, 256, 2), jnp.int32)
    t2 = lax.bitcast_convert_type(table2.reshape(NP, 128, 2), jnp.int32)
    return _make_sc_gather_pair()(t1, src_idx, t2, dst_idx)


def _gather_n256(table, idx):
    return _make_sc_gather(NP, 256, 64)(table, idx)


NBS = E_PER_TILE // 128


@functools.lru_cache(maxsize=None)
def _make_sc_scatter():
    @functools.partial(
        pl.kernel,
        mesh=plsc.VectorSubcoreMesh(core_axis_name="c", subcore_axis_name="s"),
        out_type=jax.ShapeDtypeStruct((2, 4, NP, 128), jnp.float32),
        scratch_types=[
            pltpu.VMEM((NBS, 128), jnp.int32),
            pltpu.VMEM((128, 128), jnp.float32),
            pltpu.VMEM((128, 128), jnp.float32),
            pltpu.VMEM_SHARED((NP, 128), jnp.float32),
        ] + [pltpu.SemaphoreType.DMA] * 5,
    )
    def scatter_body(dst_hbm, vals_hbm, zeros_hbm, out_hbm, idx_v, va, vb,
                     table_sh, l0, l1, a0, a1, isem):
        cid = lax.axis_index("c")
        sid = lax.axis_index("s")
        wid = sid * 2 + cid
        base = wid * E_PER_TILE
        row0 = sid * NROWS_PER_TILE
        ih = [
            pltpu.async_copy(dst_hbm.at[pl.ds(base + b * 128, 128)],
                             idx_v.at[b], isem)
            for b in range(NBS)
        ]
        for h in ih:
            h.wait()
        vbufs = [va, vb]
        lsems = [l0, l1]
        asems = [a0, a1]
        for c in range(4):
            pltpu.sync_copy(zeros_hbm, table_sh.at[pl.ds(row0,
                                                         NROWS_PER_TILE)])
            plsc.subcore_barrier()
            lh = [None] * NBS
            ah = [None] * NBS
            for b in range(NBS + 1):
                if b < NBS:
                    if b >= 2:
                        ah[b - 2].wait()
                    lh[b] = pltpu.async_copy(
                        vals_hbm.at[c, pl.ds(base + b * 128, 128)],
                        vbufs[b % 2], lsems[b % 2])
                if b >= 1:
                    lh[b - 1].wait()
                    ah[b - 1] = pltpu.async_copy(
                        vbufs[(b - 1) % 2], table_sh.at[idx_v.at[b - 1]],
                        asems[(b - 1) % 2], add=True)
            ah[NBS - 2].wait()
            ah[NBS - 1].wait()
            plsc.subcore_barrier()
            pltpu.sync_copy(
                table_sh.at[pl.ds(row0, NROWS_PER_TILE)],
                out_hbm.at[cid, c, pl.ds(row0, NROWS_PER_TILE)],
            )
            plsc.subcore_barrier()

    return scatter_body


def _scatter_k(dstp, vals4, zeros_n):
    return _make_sc_scatter()(dstp, vals4, zeros_n)


def _xplor(bl):
    r_on, r_cut = 7.5, 8.0
    r2 = bl * bl
    ron2 = r_on * r_on
    rc2 = r_cut * r_cut
    smooth = ((rc2 - r2) ** 2 * (rc2 + 2.0 * r2 - 3.0 * ron2)) / (rc2 - ron2) ** 3
    return jnp.where(bl < r_on, 1.0, jnp.where(bl < r_cut, smooth, 0.0))


def _layernorm(x, g, b, eps=1e-5):
    mu = jnp.mean(x, axis=-1, keepdims=True)
    var = jnp.mean((x - mu) ** 2, axis=-1, keepdims=True)
    return (x - mu) / jnp.sqrt(var + eps) * g + b


def _silu(x):
    return x * jax.nn.sigmoid(x)


def _rbf_body(r_ref, y_ref):
    r = r_ref[...]
    bl = jnp.sqrt(jnp.sum(r * r, axis=1, keepdims=True))
    centers = lax.broadcasted_iota(jnp.int32, (1, H), 1).astype(
        jnp.float32) * (8.0 / (H - 1))
    gamma = 1.0 / (8.0 / (H - 1)) ** 2
    y_ref[...] = jnp.exp(-gamma * (bl - centers) ** 2)


def _rbf(rp):
    return pl.pallas_call(
        _rbf_body,
        grid=(EP // BE,),
        in_specs=[pl.BlockSpec((BE, 8), lambda i: (i, 0))],
        out_specs=pl.BlockSpec((BE, H), lambda i: (i, 0)),
        out_shape=jax.ShapeDtypeStruct((EP, H), jnp.float32),
    )(rp)


def _nodelin_body(x_ref, w_ref, b_ref, o1_ref, o2_ref, o3_ref):
    acc = jnp.dot(x_ref[...], w_ref[...], preferred_element_type=jnp.float32)
    acc = acc + b_ref[...]
    o1_ref[...] = acc[:, :512].astype(jnp.bfloat16)
    o2_ref[...] = acc[:, 512:768].astype(jnp.bfloat16)
    o3_ref[...] = acc[:, 768:]


def _nodelin(x, wcat, bcat):
    return pl.pallas_call(
        _nodelin_body,
        grid=(NP // BN,),
        in_specs=[
            pl.BlockSpec((BN, H), lambda i: (i, 0)),
            pl.BlockSpec((H, 4 * H), lambda i: (0, 0)),
            pl.BlockSpec((1, 4 * H), lambda i: (0, 0)),
        ],
        out_specs=(
            pl.BlockSpec((BN, 512), lambda i: (i, 0)),
            pl.BlockSpec((BN, 256), lambda i: (i, 0)),
            pl.BlockSpec((BN, 256), lambda i: (i, 0)),
        ),
        out_shape=(
            jax.ShapeDtypeStruct((NP, 512), jnp.bfloat16),
            jax.ShapeDtypeStruct((NP, 256), jnp.bfloat16),
            jax.ShapeDtypeStruct((NP, 256), jnp.float32),
        ),
    )(x, wcat, bcat)


def _edge_body(y_ref, g1_ref, g2_ref, r_ref, weg_ref, beg_ref, ge_ref, be_ref,
               ynew_ref, vals_ref):
    y = y_ref[...]
    g1w = g1_ref[...]
    g2w = g2_ref[...]
    mask = jnp.int32(-65536)
    lo1 = lax.bitcast_convert_type(g1w << 16, jnp.float32)
    hi1 = lax.bitcast_convert_type(g1w & mask, jnp.float32)
    lo2 = lax.bitcast_convert_type(g2w << 16, jnp.float32)
    hi2 = lax.bitcast_convert_type(g2w & mask, jnp.float32)
    e_src = jnp.concatenate([lo1[:, :128], hi1[:, :128]], axis=1)
    bh = jnp.concatenate([lo1[:, 128:], hi1[:, 128:]], axis=1)
    e_dst = jnp.concatenate([lo2, hi2], axis=1)
    m = e_src + e_dst + beg_ref[...]
    m = m + jnp.dot(y, weg_ref[...], preferred_element_type=jnp.float32)
    r = r_ref[...]
    bl = jnp.sqrt(jnp.sum(r * r, axis=1, keepdims=True))
    cval = _xplor(bl)
    sig = jax.nn.sigmoid(m) * cval
    u = bh * sig
    vals_ref[0] = sig[:, :128]
    vals_ref[1] = sig[:, 128:]
    vals_ref[2] = u[:, :128]
    vals_ref[3] = u[:, 128:]
    ynew_ref[...] = y + _silu(_layernorm(m, ge_ref[...], be_ref[...]))


def _edge(y, g1, g2, rp, weg, beg, gel, bel):
    return pl.pallas_call(
        _edge_body,
        grid=(EP // BE,),
        in_specs=[
            pl.BlockSpec((BE, H), lambda i: (i, 0)),
            pl.BlockSpec((BE, 256), lambda i: (i, 0)),
            pl.BlockSpec((BE, 128), lambda i: (i, 0)),
            pl.BlockSpec((BE, 8), lambda i: (i, 0)),
            pl.BlockSpec((H, H), lambda i: (0, 0)),
            pl.BlockSpec((1, H), lambda i: (0, 0)),
            pl.BlockSpec((1, H), lambda i: (0, 0)),
            pl.BlockSpec((1, H), lambda i: (0, 0)),
        ],
        out_specs=(
            pl.BlockSpec((BE, H), lambda i: (i, 0)),
            pl.BlockSpec((4, BE, 128), lambda i: (0, i, 0)),
        ),
        out_shape=(
            jax.ShapeDtypeStruct((EP, H), jnp.float32),
            jax.ShapeDtypeStruct((4, EP, 128), jnp.float32),
        ),
    )(y, g1, g2, rp, weg, beg, gel, bel)


def _nodeupd_body(x_ref, o3_ref, s_ref, gn_ref, bn_ref, out_ref):
    s = s_ref[...]
    ssum = s[0] + s[1]
    sum_sigma = jnp.concatenate([ssum[0], ssum[1]], axis=1)
    sum_h = jnp.concatenate([ssum[2], ssum[3]], axis=1)
    h = sum_h / (sum_sigma + 1e-6)
    xu = o3_ref[...] + h
    out_ref[...] = x_ref[...] + _silu(_layernorm(xu, gn_ref[...], bn_ref[...]))


def _nodeupd(x, o3, s, gnl, bnl):
    return pl.pallas_call(
        _nodeupd_body,
        grid=(NP // BN,),
        in_specs=[
            pl.BlockSpec((BN, H), lambda i: (i, 0)),
            pl.BlockSpec((BN, H), lambda i: (i, 0)),
            pl.BlockSpec((2, 4, BN, 128), lambda i: (0, 0, i, 0)),
            pl.BlockSpec((1, H), lambda i: (0, 0)),
            pl.BlockSpec((1, H), lambda i: (0, 0)),
        ],
        out_specs=pl.BlockSpec((BN, H), lambda i: (i, 0)),
        out_shape=jax.ShapeDtypeStruct((NP, H), jnp.float32),
    )(x, o3, s, gnl, bnl)


def _readout_body(x_ref, w_ref, out_ref):
    e = jnp.dot(x_ref[...], w_ref[...], preferred_element_type=jnp.float32)
    rows = lax.broadcasted_iota(jnp.int32, (NP, 128), 0)
    e = jnp.where(rows < N, e, 0.0)
    out_ref[...] = jnp.reshape(jnp.sum(e) / N, (1, 1))


def _readout(x, wfc_pad):
    return pl.pallas_call(
        _readout_body,
        in_specs=[
            pl.BlockSpec((NP, H), lambda: (0, 0)),
            pl.BlockSpec((H, 128), lambda: (0, 0)),
        ],
        out_specs=pl.BlockSpec((1, 1), lambda: (0, 0)),
        out_shape=jax.ShapeDtypeStruct((1, 1), jnp.float32),
    )(x, wfc_pad)


def kernel(atomic_number, edge_index, r, atom_emb, Wsg, bsg, Wdg, bdg, Weg,
           beg, Wsu, bsu, Wdu, bdu, gn, bn, ge, be, Wfc, bfc):
    src = edge_index[0].astype(jnp.int32)
    dst = edge_index[1].astype(jnp.int32)
    srcp = jnp.pad(src, (0, EP - E))
    dstp = jnp.pad(dst, (0, EP - E), constant_values=PAD_DST)
    rp = jnp.pad(r, ((0, EP - E), (0, 5)))
    anp = jnp.pad(atomic_number.astype(jnp.int32), (0, NP - N))
    zeros_n = jnp.zeros((NROWS_PER_TILE, 128), jnp.float32)

    x = _gather_n256(atom_emb, anp)
    y = _rbf(rp)

    for l in range(L):
        wcat = jnp.concatenate([Wsg[l][:, _SIGINV], Wsu[l][:, _SIGINV],
                                Wdg[l][:, _SIGINV], Wdu[l]], axis=1)
        bcat = jnp.concatenate([bsg[l][_SIGINV], bsu[l][_SIGINV],
                                bdg[l][_SIGINV], bdu[l]])[None, :]
        o1, o2, o3 = _nodelin(x, wcat, bcat)
        g1, g2 = _gather_edges(o1, srcp, o2, dstp)
        y, vals4 = _edge(y, g1, g2, rp, Weg[l], beg[l][None, :],
                         ge[l][None, :], be[l][None, :])
        s = _scatter_k(dstp, vals4, zeros_n)
        x = _nodeupd(x, o3, s, gn[l][None, :], bn[l][None, :])

    wfc_pad = jnp.pad(Wfc, ((0, 0), (0, 127)))
    out = _readout(x, wfc_pad)
    return out[0, 0] + bfc[0]


# speedup vs baseline: 2.0974x; 1.0581x over previous
import functools

import numpy as np

import jax
import jax.numpy as jnp
from jax import lax
from jax.experimental import pallas as pl
from jax.experimental.pallas import tpu as pltpu
from jax.experimental.pallas import tpu_sc as plsc

N = 10000
E = 160000
H = 256
L = 3

NP = 10240
EP = 163840
NW = 32
E_PER_TILE = EP // NW
NROWS_PER_TILE = NP // 16
PAD_DST = N + 16
BE = 512
BN = 256

_SIG = np.concatenate([np.arange(128) * 2, np.arange(128) * 2 + 1])
_SIGINV = np.argsort(_SIG)

class _GatherPipe:
    def __init__(self, table_hbm, idx_hbm, out_hbm, idx_all, bufs, gsems,
                 wsems, base, npt, bg):
        pltpu.sync_copy(idx_hbm.at[pl.ds(base, npt)], idx_all)
        self.t, self.o, self.idx = table_hbm, out_hbm, idx_all
        self.bufs, self.gs, self.ws = bufs, gsems, wsems
        self.base, self.bg = base, bg
        self.nblk = npt // bg
        self.gh = [None] * self.nblk
        self.wh = [None] * self.nblk

    def step(self, b):
        if b < self.nblk:
            if b >= 2:
                self.wh[b - 2].wait()
            self.gh[b] = pltpu.async_copy(
                self.t.at[self.idx.at[pl.ds(b * self.bg, self.bg)]],
                self.bufs[b % 2], self.gs[b % 2])
        if 1 <= b <= self.nblk:
            self.gh[b - 1].wait()
            self.wh[b - 1] = pltpu.async_copy(
                self.bufs[(b - 1) % 2],
                self.o.at[pl.ds(self.base + (b - 1) * self.bg, self.bg)],
                self.ws[(b - 1) % 2])

    def drain(self):
        self.wh[self.nblk - 2].wait()
        self.wh[self.nblk - 1].wait()


def _pipelined_gather(table_hbm, idx_hbm, out_hbm, idx_all, bufs, gsems,
                      wsems, base, npt, bg):
    pipe = _GatherPipe(table_hbm, idx_hbm, out_hbm, idx_all, bufs, gsems,
                       wsems, base, npt, bg)
    for b in range(pipe.nblk + 1):
        pipe.step(b)
    pipe.drain()


@functools.lru_cache(maxsize=None)
def _make_sc_gather(n_out, n_cols, bg, dtype=jnp.float32):
    n_per_tile = n_out // NW
    nblk = n_per_tile // bg

    @functools.partial(
        pl.kernel,
        mesh=plsc.VectorSubcoreMesh(core_axis_name="c", subcore_axis_name="s"),
        out_type=jax.ShapeDtypeStruct((n_out, n_cols), dtype),
        scratch_types=[
            pltpu.VMEM((n_per_tile,), jnp.int32),
            pltpu.VMEM((bg, n_cols), dtype),
            pltpu.VMEM((bg, n_cols), dtype),
        ] + [pltpu.SemaphoreType.DMA] * 4,
    )
    def gather_k(t1, i1, o1, idx1, ba, bb, s0, s1, s2, s3):
        cid = lax.axis_index("c")
        sid = lax.axis_index("s")
        wid = sid * 2 + cid
        _pipelined_gather(t1, i1, o1, idx1, [ba, bb], [s0, s1], [s2, s3],
                          wid * n_per_tile, n_per_tile, bg)

    return gather_k


EH = EP // 2
EH_PER_TILE = EH // NW


@functools.lru_cache(maxsize=None)
def _make_sc_gather_pair(off):
    bg = 128
    nblk = EH_PER_TILE // bg

    @functools.partial(
        pl.kernel,
        mesh=plsc.VectorSubcoreMesh(core_axis_name="c", subcore_axis_name="s"),
        out_type=(
            jax.ShapeDtypeStruct((EH, 256), jnp.int32),
            jax.ShapeDtypeStruct((EH, 128), jnp.int32),
        ),
        scratch_types=[
            pltpu.VMEM((EH_PER_TILE,), jnp.int32),
            pltpu.VMEM((EH_PER_TILE,), jnp.int32),
            pltpu.VMEM((bg, 256), jnp.int32),
            pltpu.VMEM((bg, 256), jnp.int32),
            pltpu.VMEM((bg, 128), jnp.int32),
            pltpu.VMEM((bg, 128), jnp.int32),
        ] + [pltpu.SemaphoreType.DMA] * 8,
    )
    def gather_k(t1, i1, t2, i2, o1, o2, idx1, idx2, b1a, b1b, b2a, b2b,
                 s0, s1, s2, s3, s4, s5, s6, s7):
        cid = lax.axis_index("c")
        sid = lax.axis_index("s")
        wid = sid * 2 + cid
        ibase = off + wid * EH_PER_TILE
        obase = wid * EH_PER_TILE
        p1 = _GatherPipe(t1, i1, o1, idx1, [b1a, b1b], [s0, s1], [s2, s3],
                         ibase, EH_PER_TILE, bg)
        p1.base = obase
        p2 = _GatherPipe(t2, i2, o2, idx2, [b2a, b2b], [s4, s5], [s6, s7],
                         ibase, EH_PER_TILE, bg)
        p2.base = obase
        for b in range(nblk + 1):
            p1.step(b)
            p2.step(b)
        p1.drain()
        p2.drain()

    return gather_k


def _gather_edges(table1, src_idx, table2, dst_idx, off):
    t1 = lax.bitcast_convert_type(table1.reshape(NP, 256, 2), jnp.int32)
    t2 = lax.bitcast_convert_type(table2.reshape(NP, 128, 2), jnp.int32)
    return _make_sc_gather_pair(off)(t1, src_idx, t2, dst_idx)


def _gather_n256(table, idx):
    return _make_sc_gather(NP, 256, 64)(table, idx)


NBS = EH_PER_TILE // 128


@functools.lru_cache(maxsize=None)
def _make_sc_scatter(off):
    @functools.partial(
        pl.kernel,
        mesh=plsc.VectorSubcoreMesh(core_axis_name="c", subcore_axis_name="s"),
        out_type=jax.ShapeDtypeStruct((2, 4, NP, 128), jnp.float32),
        scratch_types=[
            pltpu.VMEM((NBS, 128), jnp.int32),
            pltpu.VMEM((128, 128), jnp.float32),
            pltpu.VMEM((128, 128), jnp.float32),
            pltpu.VMEM_SHARED((NP, 128), jnp.float32),
        ] + [pltpu.SemaphoreType.DMA] * 5,
    )
    def scatter_body(dst_hbm, vals_hbm, zeros_hbm, out_hbm, idx_v, va, vb,
                     table_sh, l0, l1, a0, a1, isem):
        cid = lax.axis_index("c")
        sid = lax.axis_index("s")
        wid = sid * 2 + cid
        base = wid * EH_PER_TILE
        row0 = sid * NROWS_PER_TILE
        ih = [
            pltpu.async_copy(dst_hbm.at[pl.ds(off + base + b * 128, 128)],
                             idx_v.at[b], isem)
            for b in range(NBS)
        ]
        for h in ih:
            h.wait()
        vbufs = [va, vb]
        lsems = [l0, l1]
        asems = [a0, a1]
        for c in range(4):
            pltpu.sync_copy(zeros_hbm, table_sh.at[pl.ds(row0,
                                                         NROWS_PER_TILE)])
            plsc.subcore_barrier()
            lh = [None] * NBS
            ah = [None] * NBS
            for b in range(NBS + 1):
                if b < NBS:
                    if b >= 2:
                        ah[b - 2].wait()
                    lh[b] = pltpu.async_copy(
                        vals_hbm.at[c, pl.ds(base + b * 128, 128)],
                        vbufs[b % 2], lsems[b % 2])
                if b >= 1:
                    lh[b - 1].wait()
                    ah[b - 1] = pltpu.async_copy(
                        vbufs[(b - 1) % 2], table_sh.at[idx_v.at[b - 1]],
                        asems[(b - 1) % 2], add=True)
            ah[NBS - 2].wait()
            ah[NBS - 1].wait()
            plsc.subcore_barrier()
            pltpu.sync_copy(
                table_sh.at[pl.ds(row0, NROWS_PER_TILE)],
                out_hbm.at[cid, c, pl.ds(row0, NROWS_PER_TILE)],
            )
            plsc.subcore_barrier()

    return scatter_body


def _scatter_k(dstp, vals4, zeros_n, off):
    return _make_sc_scatter(off)(dstp, vals4, zeros_n)


def _xplor(bl):
    r_on, r_cut = 7.5, 8.0
    r2 = bl * bl
    ron2 = r_on * r_on
    rc2 = r_cut * r_cut
    smooth = ((rc2 - r2) ** 2 * (rc2 + 2.0 * r2 - 3.0 * ron2)) / (rc2 - ron2) ** 3
    return jnp.where(bl < r_on, 1.0, jnp.where(bl < r_cut, smooth, 0.0))


def _layernorm(x, g, b, eps=1e-5):
    mu = jnp.mean(x, axis=-1, keepdims=True)
    var = jnp.mean((x - mu) ** 2, axis=-1, keepdims=True)
    return (x - mu) / jnp.sqrt(var + eps) * g + b


def _silu(x):
    return x * jax.nn.sigmoid(x)


def _rbf_body(r_ref, y_ref):
    r = r_ref[...]
    bl = jnp.sqrt(jnp.sum(r * r, axis=1, keepdims=True))
    centers = lax.broadcasted_iota(jnp.int32, (1, H), 1).astype(
        jnp.float32) * (8.0 / (H - 1))
    gamma = 1.0 / (8.0 / (H - 1)) ** 2
    y_ref[...] = jnp.exp(-gamma * (bl - centers) ** 2)


def _rbf(rp, off):
    blk0 = off // BE
    return pl.pallas_call(
        _rbf_body,
        grid=(EH // BE,),
        in_specs=[pl.BlockSpec((BE, 8), lambda i: (i + blk0, 0))],
        out_specs=pl.BlockSpec((BE, H), lambda i: (i, 0)),
        out_shape=jax.ShapeDtypeStruct((EH, H), jnp.float32),
    )(rp)


def _nodelin_body(x_ref, w_ref, b_ref, o1_ref, o2_ref, o3_ref):
    acc = jnp.dot(x_ref[...], w_ref[...], preferred_element_type=jnp.float32)
    acc = acc + b_ref[...]
    o1_ref[...] = acc[:, :512].astype(jnp.bfloat16)
    o2_ref[...] = acc[:, 512:768].astype(jnp.bfloat16)
    o3_ref[...] = acc[:, 768:]


def _nodelin(x, wcat, bcat):
    return pl.pallas_call(
        _nodelin_body,
        grid=(NP // BN,),
        in_specs=[
            pl.BlockSpec((BN, H), lambda i: (i, 0)),
            pl.BlockSpec((H, 4 * H), lambda i: (0, 0)),
            pl.BlockSpec((1, 4 * H), lambda i: (0, 0)),
        ],
        out_specs=(
            pl.BlockSpec((BN, 512), lambda i: (i, 0)),
            pl.BlockSpec((BN, 256), lambda i: (i, 0)),
            pl.BlockSpec((BN, 256), lambda i: (i, 0)),
        ),
        out_shape=(
            jax.ShapeDtypeStruct((NP, 512), jnp.bfloat16),
            jax.ShapeDtypeStruct((NP, 256), jnp.bfloat16),
            jax.ShapeDtypeStruct((NP, 256), jnp.float32),
        ),
    )(x, wcat, bcat)


def _edge_body(y_ref, g1_ref, g2_ref, r_ref, weg_ref, beg_ref, ge_ref, be_ref,
               ynew_ref, vals_ref):
    y = y_ref[...]
    g1w = g1_ref[...]
    g2w = g2_ref[...]
    mask = jnp.int32(-65536)
    lo1 = lax.bitcast_convert_type(g1w << 16, jnp.float32)
    hi1 = lax.bitcast_convert_type(g1w & mask, jnp.float32)
    lo2 = lax.bitcast_convert_type(g2w << 16, jnp.float32)
    hi2 = lax.bitcast_convert_type(g2w & mask, jnp.float32)
    e_src = jnp.concatenate([lo1[:, :128], hi1[:, :128]], axis=1)
    bh = jnp.concatenate([lo1[:, 128:], hi1[:, 128:]], axis=1)
    e_dst = jnp.concatenate([lo2, hi2], axis=1)
    m = e_src + e_dst + beg_ref[...]
    m = m + jnp.dot(y, weg_ref[...], preferred_element_type=jnp.float32)
    r = r_ref[...]
    bl = jnp.sqrt(jnp.sum(r * r, axis=1, keepdims=True))
    cval = _xplor(bl)
    sig = jax.nn.sigmoid(m) * cval
    u = bh * sig
    vals_ref[0] = sig[:, :128]
    vals_ref[1] = sig[:, 128:]
    vals_ref[2] = u[:, :128]
    vals_ref[3] = u[:, 128:]
    ynew_ref[...] = y + _silu(_layernorm(m, ge_ref[...], be_ref[...]))


def _edge(y, g1, g2, rp, weg, beg, gel, bel, off):
    blk0 = off // BE
    return pl.pallas_call(
        _edge_body,
        grid=(EH // BE,),
        in_specs=[
            pl.BlockSpec((BE, H), lambda i: (i, 0)),
            pl.BlockSpec((BE, 256), lambda i: (i, 0)),
            pl.BlockSpec((BE, 128), lambda i: (i, 0)),
            pl.BlockSpec((BE, 8), lambda i: (i + blk0, 0)),
            pl.BlockSpec((H, H), lambda i: (0, 0)),
            pl.BlockSpec((1, H), lambda i: (0, 0)),
            pl.BlockSpec((1, H), lambda i: (0, 0)),
            pl.BlockSpec((1, H), lambda i: (0, 0)),
        ],
        out_specs=(
            pl.BlockSpec((BE, H), lambda i: (i, 0)),
            pl.BlockSpec((4, BE, 128), lambda i: (0, i, 0)),
        ),
        out_shape=(
            jax.ShapeDtypeStruct((EH, H), jnp.float32),
            jax.ShapeDtypeStruct((4, EH, 128), jnp.float32),
        ),
    )(y, g1, g2, rp, weg, beg, gel, bel)


def _nodeupd_body(x_ref, o3_ref, sa_ref, sb_ref, gn_ref, bn_ref, out_ref):
    sa = sa_ref[...]
    sb = sb_ref[...]
    ssum = sa[0] + sa[1] + sb[0] + sb[1]
    sum_sigma = jnp.concatenate([ssum[0], ssum[1]], axis=1)
    sum_h = jnp.concatenate([ssum[2], ssum[3]], axis=1)
    h = sum_h / (sum_sigma + 1e-6)
    xu = o3_ref[...] + h
    out_ref[...] = x_ref[...] + _silu(_layernorm(xu, gn_ref[...], bn_ref[...]))


def _nodeupd(x, o3, sa, sb, gnl, bnl):
    return pl.pallas_call(
        _nodeupd_body,
        grid=(NP // BN,),
        in_specs=[
            pl.BlockSpec((BN, H), lambda i: (i, 0)),
            pl.BlockSpec((BN, H), lambda i: (i, 0)),
            pl.BlockSpec((2, 4, BN, 128), lambda i: (0, 0, i, 0)),
            pl.BlockSpec((2, 4, BN, 128), lambda i: (0, 0, i, 0)),
            pl.BlockSpec((1, H), lambda i: (0, 0)),
            pl.BlockSpec((1, H), lambda i: (0, 0)),
        ],
        out_specs=pl.BlockSpec((BN, H), lambda i: (i, 0)),
        out_shape=jax.ShapeDtypeStruct((NP, H), jnp.float32),
    )(x, o3, sa, sb, gnl, bnl)


def _readout_body(x_ref, w_ref, out_ref):
    e = jnp.dot(x_ref[...], w_ref[...], preferred_element_type=jnp.float32)
    rows = lax.broadcasted_iota(jnp.int32, (NP, 128), 0)
    e = jnp.where(rows < N, e, 0.0)
    out_ref[...] = jnp.reshape(jnp.sum(e) / N, (1, 1))


def _readout(x, wfc_pad):
    return pl.pallas_call(
        _readout_body,
        in_specs=[
            pl.BlockSpec((NP, H), lambda: (0, 0)),
            pl.BlockSpec((H, 128), lambda: (0, 0)),
        ],
        out_specs=pl.BlockSpec((1, 1), lambda: (0, 0)),
        out_shape=jax.ShapeDtypeStruct((1, 1), jnp.float32),
    )(x, wfc_pad)


def kernel(atomic_number, edge_index, r, atom_emb, Wsg, bsg, Wdg, bdg, Weg,
           beg, Wsu, bsu, Wdu, bdu, gn, bn, ge, be, Wfc, bfc):
    src = edge_index[0].astype(jnp.int32)
    dst = edge_index[1].astype(jnp.int32)
    srcp = jnp.pad(src, (0, EP - E))
    dstp = jnp.pad(dst, (0, EP - E), constant_values=PAD_DST)
    rp = jnp.pad(r, ((0, EP - E), (0, 5)))
    anp = jnp.pad(atomic_number.astype(jnp.int32), (0, NP - N))
    zeros_n = jnp.zeros((NROWS_PER_TILE, 128), jnp.float32)

    x = _gather_n256(atom_emb, anp)
    ya = _rbf(rp, 0)
    yb = _rbf(rp, EH)

    for l in range(L):
        wcat = jnp.concatenate([Wsg[l][:, _SIGINV], Wsu[l][:, _SIGINV],
                                Wdg[l][:, _SIGINV], Wdu[l]], axis=1)
        bcat = jnp.concatenate([bsg[l][_SIGINV], bsu[l][_SIGINV],
                                bdg[l][_SIGINV], bdu[l]])[None, :]
        o1, o2, o3 = _nodelin(x, wcat, bcat)
        wegl = Weg[l]
        begl = beg[l][None, :]
        gel = ge[l][None, :]
        bel = be[l][None, :]
        g1a, g2a = _gather_edges(o1, srcp, o2, dstp, 0)
        g1b, g2b = _gather_edges(o1, srcp, o2, dstp, EH)
        ya, vals4a = _edge(ya, g1a, g2a, rp, wegl, begl, gel, bel, 0)
        sa = _scatter_k(dstp, vals4a, zeros_n, 0)
        yb, vals4b = _edge(yb, g1b, g2b, rp, wegl, begl, gel, bel, EH)
        sb = _scatter_k(dstp, vals4b, zeros_n, EH)
        x = _nodeupd(x, o3, sa, sb, gn[l][None, :], bn[l][None, :])

    wfc_pad = jnp.pad(Wfc, ((0, 0), (0, 127)))
    out = _readout(x, wfc_pad)
    return out[0, 0] + bfc[0]


# speedup vs baseline: 2.1723x; 1.0357x over previous
import functools

import numpy as np

import jax
import jax.numpy as jnp
from jax import lax
from jax.experimental import pallas as pl
from jax.experimental.pallas import tpu as pltpu
from jax.experimental.pallas import tpu_sc as plsc

N = 10000
E = 160000
H = 256
L = 3

NP = 10240
EP = 163840
NW = 32
E_PER_TILE = EP // NW
NROWS_PER_TILE = NP // 16
PAD_DST = N + 16
BE = 512
BN = 256

_SIG = np.concatenate([np.arange(128) * 2, np.arange(128) * 2 + 1])
_SIGINV = np.argsort(_SIG)

class _GatherPipe:
    def __init__(self, table_hbm, idx_hbm, out_hbm, idx_all, bufs, gsems,
                 wsems, base, npt, bg):
        pltpu.sync_copy(idx_hbm.at[pl.ds(base, npt)], idx_all)
        self.t, self.o, self.idx = table_hbm, out_hbm, idx_all
        self.bufs, self.gs, self.ws = bufs, gsems, wsems
        self.base, self.bg = base, bg
        self.nblk = npt // bg
        self.gh = [None] * self.nblk
        self.wh = [None] * self.nblk

    def step(self, b):
        if b < self.nblk:
            if b >= 2:
                self.wh[b - 2].wait()
            self.gh[b] = pltpu.async_copy(
                self.t.at[self.idx.at[pl.ds(b * self.bg, self.bg)]],
                self.bufs[b % 2], self.gs[b % 2])
        if 1 <= b <= self.nblk:
            self.gh[b - 1].wait()
            self.wh[b - 1] = pltpu.async_copy(
                self.bufs[(b - 1) % 2],
                self.o.at[pl.ds(self.base + (b - 1) * self.bg, self.bg)],
                self.ws[(b - 1) % 2])

    def drain(self):
        self.wh[self.nblk - 2].wait()
        self.wh[self.nblk - 1].wait()


def _pipelined_gather(table_hbm, idx_hbm, out_hbm, idx_all, bufs, gsems,
                      wsems, base, npt, bg):
    pipe = _GatherPipe(table_hbm, idx_hbm, out_hbm, idx_all, bufs, gsems,
                       wsems, base, npt, bg)
    for b in range(pipe.nblk + 1):
        pipe.step(b)
    pipe.drain()


@functools.lru_cache(maxsize=None)
def _make_sc_gather(n_out, n_cols, bg, dtype=jnp.float32):
    n_per_tile = n_out // NW
    nblk = n_per_tile // bg

    @functools.partial(
        pl.kernel,
        mesh=plsc.VectorSubcoreMesh(core_axis_name="c", subcore_axis_name="s"),
        out_type=jax.ShapeDtypeStruct((n_out, n_cols), dtype),
        scratch_types=[
            pltpu.VMEM((n_per_tile,), jnp.int32),
            pltpu.VMEM((bg, n_cols), dtype),
            pltpu.VMEM((bg, n_cols), dtype),
        ] + [pltpu.SemaphoreType.DMA] * 4,
    )
    def gather_k(t1, i1, o1, idx1, ba, bb, s0, s1, s2, s3):
        cid = lax.axis_index("c")
        sid = lax.axis_index("s")
        wid = sid * 2 + cid
        _pipelined_gather(t1, i1, o1, idx1, [ba, bb], [s0, s1], [s2, s3],
                          wid * n_per_tile, n_per_tile, bg)

    return gather_k


EH = EP // 2
EH_PER_TILE = EH // NW


@functools.lru_cache(maxsize=None)
def _make_sc_gather_pair(off):
    bg = 128
    nblk = EH_PER_TILE // bg

    @functools.partial(
        pl.kernel,
        mesh=plsc.VectorSubcoreMesh(core_axis_name="c", subcore_axis_name="s"),
        out_type=(
            jax.ShapeDtypeStruct((EH, 256), jnp.int32),
            jax.ShapeDtypeStruct((EH, 128), jnp.int32),
        ),
        scratch_types=[
            pltpu.VMEM((EH_PER_TILE,), jnp.int32),
            pltpu.VMEM((EH_PER_TILE,), jnp.int32),
            pltpu.VMEM((bg, 256), jnp.int32),
            pltpu.VMEM((bg, 256), jnp.int32),
            pltpu.VMEM((bg, 128), jnp.int32),
            pltpu.VMEM((bg, 128), jnp.int32),
        ] + [pltpu.SemaphoreType.DMA] * 8,
    )
    def gather_k(t1, i1, t2, i2, o1, o2, idx1, idx2, b1a, b1b, b2a, b2b,
                 s0, s1, s2, s3, s4, s5, s6, s7):
        cid = lax.axis_index("c")
        sid = lax.axis_index("s")
        wid = sid * 2 + cid
        ibase = off + wid * EH_PER_TILE
        obase = wid * EH_PER_TILE
        p1 = _GatherPipe(t1, i1, o1, idx1, [b1a, b1b], [s0, s1], [s2, s3],
                         ibase, EH_PER_TILE, bg)
        p1.base = obase
        p2 = _GatherPipe(t2, i2, o2, idx2, [b2a, b2b], [s4, s5], [s6, s7],
                         ibase, EH_PER_TILE, bg)
        p2.base = obase
        for b in range(nblk + 1):
            p1.step(b)
            p2.step(b)
        p1.drain()
        p2.drain()

    return gather_k


def _gather_edges(table1, src_idx, table2, dst_idx, off):
    t1 = lax.bitcast_convert_type(table1.reshape(NP, 256, 2), jnp.int32)
    t2 = lax.bitcast_convert_type(table2.reshape(NP, 128, 2), jnp.int32)
    return _make_sc_gather_pair(off)(t1, src_idx, t2, dst_idx)


def _gather_n256(table, idx):
    return _make_sc_gather(NP, 256, 64)(table, idx)


NBS = EH_PER_TILE // 128


@functools.lru_cache(maxsize=None)
def _make_sc_scatter(off):
    @functools.partial(
        pl.kernel,
        mesh=plsc.VectorSubcoreMesh(core_axis_name="c", subcore_axis_name="s"),
        out_type=jax.ShapeDtypeStruct((2, 4, NP, 128), jnp.float32),
        scratch_types=[
            pltpu.VMEM((NBS, 128), jnp.int32),
            pltpu.VMEM((128, 128), jnp.float32),
            pltpu.VMEM((128, 128), jnp.float32),
            pltpu.VMEM_SHARED((NP, 128), jnp.float32),
        ] + [pltpu.SemaphoreType.DMA] * 5,
    )
    def scatter_body(dst_hbm, vals_hbm, zeros_hbm, out_hbm, idx_v, va, vb,
                     table_sh, l0, l1, a0, a1, isem):
        cid = lax.axis_index("c")
        sid = lax.axis_index("s")
        wid = sid * 2 + cid
        base = wid * EH_PER_TILE
        row0 = sid * NROWS_PER_TILE
        ih = [
            pltpu.async_copy(dst_hbm.at[pl.ds(off + base + b * 128, 128)],
                             idx_v.at[b], isem)
            for b in range(NBS)
        ]
        for h in ih:
            h.wait()
        vbufs = [va, vb]
        lsems = [l0, l1]
        asems = [a0, a1]
        for c in range(4):
            pltpu.sync_copy(zeros_hbm, table_sh.at[pl.ds(row0,
                                                         NROWS_PER_TILE)])
            plsc.subcore_barrier()
            lh = [None] * NBS
            ah = [None] * NBS
            for b in range(NBS + 1):
                if b < NBS:
                    if b >= 2:
                        ah[b - 2].wait()
                    lh[b] = pltpu.async_copy(
                        vals_hbm.at[c, pl.ds(base + b * 128, 128)],
                        vbufs[b % 2], lsems[b % 2])
                if b >= 1:
                    lh[b - 1].wait()
                    ah[b - 1] = pltpu.async_copy(
                        vbufs[(b - 1) % 2], table_sh.at[idx_v.at[b - 1]],
                        asems[(b - 1) % 2], add=True)
            ah[NBS - 2].wait()
            ah[NBS - 1].wait()
            plsc.subcore_barrier()
            pltpu.sync_copy(
                table_sh.at[pl.ds(row0, NROWS_PER_TILE)],
                out_hbm.at[cid, c, pl.ds(row0, NROWS_PER_TILE)],
            )
            plsc.subcore_barrier()

    return scatter_body


def _scatter_k(dstp, vals4, zeros_n, off):
    return _make_sc_scatter(off)(dstp, vals4, zeros_n)


def _xplor(bl):
    r_on, r_cut = 7.5, 8.0
    r2 = bl * bl
    ron2 = r_on * r_on
    rc2 = r_cut * r_cut
    smooth = ((rc2 - r2) ** 2 * (rc2 + 2.0 * r2 - 3.0 * ron2)) / (rc2 - ron2) ** 3
    return jnp.where(bl < r_on, 1.0, jnp.where(bl < r_cut, smooth, 0.0))


def _layernorm(x, g, b, eps=1e-5):
    mu = jnp.mean(x, axis=-1, keepdims=True)
    var = jnp.mean((x - mu) ** 2, axis=-1, keepdims=True)
    return (x - mu) / jnp.sqrt(var + eps) * g + b


def _silu(x):
    return x * jax.nn.sigmoid(x)


def _rbf_body(r_ref, y_ref):
    r = r_ref[...]
    bl = jnp.sqrt(jnp.sum(r * r, axis=1, keepdims=True))
    centers = lax.broadcasted_iota(jnp.int32, (1, H), 1).astype(
        jnp.float32) * (8.0 / (H - 1))
    gamma = 1.0 / (8.0 / (H - 1)) ** 2
    y_ref[...] = jnp.exp(-gamma * (bl - centers) ** 2).astype(jnp.bfloat16)


def _rbf(rp, off):
    blk0 = off // BE
    return pl.pallas_call(
        _rbf_body,
        grid=(EH // BE,),
        in_specs=[pl.BlockSpec((BE, 8), lambda i: (i + blk0, 0))],
        out_specs=pl.BlockSpec((BE, H), lambda i: (i, 0)),
        out_shape=jax.ShapeDtypeStruct((EH, H), jnp.bfloat16),
    )(rp)


def _nodelin_body(x_ref, w_ref, b_ref, o1_ref, o2_ref, o3_ref):
    acc = jnp.dot(x_ref[...], w_ref[...], preferred_element_type=jnp.float32)
    acc = acc + b_ref[...]
    o1_ref[...] = acc[:, :512].astype(jnp.bfloat16)
    o2_ref[...] = acc[:, 512:768].astype(jnp.bfloat16)
    o3_ref[...] = acc[:, 768:]


def _nodelin(x, wcat, bcat):
    return pl.pallas_call(
        _nodelin_body,
        grid=(NP // BN,),
        in_specs=[
            pl.BlockSpec((BN, H), lambda i: (i, 0)),
            pl.BlockSpec((H, 4 * H), lambda i: (0, 0)),
            pl.BlockSpec((1, 4 * H), lambda i: (0, 0)),
        ],
        out_specs=(
            pl.BlockSpec((BN, 512), lambda i: (i, 0)),
            pl.BlockSpec((BN, 256), lambda i: (i, 0)),
            pl.BlockSpec((BN, 256), lambda i: (i, 0)),
        ),
        out_shape=(
            jax.ShapeDtypeStruct((NP, 512), jnp.bfloat16),
            jax.ShapeDtypeStruct((NP, 256), jnp.bfloat16),
            jax.ShapeDtypeStruct((NP, 256), jnp.float32),
        ),
    )(x, wcat, bcat)


def _edge_body(y_ref, g1_ref, g2_ref, r_ref, weg_ref, beg_ref, ge_ref, be_ref,
               ynew_ref, vals_ref):
    y = y_ref[...]
    y32 = y.astype(jnp.float32)
    g1w = g1_ref[...]
    g2w = g2_ref[...]
    mask = jnp.int32(-65536)
    lo1 = lax.bitcast_convert_type(g1w << 16, jnp.float32)
    hi1 = lax.bitcast_convert_type(g1w & mask, jnp.float32)
    lo2 = lax.bitcast_convert_type(g2w << 16, jnp.float32)
    hi2 = lax.bitcast_convert_type(g2w & mask, jnp.float32)
    e_src = jnp.concatenate([lo1[:, :128], hi1[:, :128]], axis=1)
    bh = jnp.concatenate([lo1[:, 128:], hi1[:, 128:]], axis=1)
    e_dst = jnp.concatenate([lo2, hi2], axis=1)
    m = e_src + e_dst + beg_ref[...]
    m = m + jnp.dot(y, weg_ref[...], preferred_element_type=jnp.float32)
    r = r_ref[...]
    bl = jnp.sqrt(jnp.sum(r * r, axis=1, keepdims=True))
    cval = _xplor(bl)
    sig = jax.nn.sigmoid(m) * cval
    u = bh * sig
    vals_ref[0] = sig[:, :128]
    vals_ref[1] = sig[:, 128:]
    vals_ref[2] = u[:, :128]
    vals_ref[3] = u[:, 128:]
    yn = y32 + _silu(_layernorm(m, ge_ref[...], be_ref[...]))
    ynew_ref[...] = yn.astype(jnp.bfloat16)


def _edge(y, g1, g2, rp, weg, beg, gel, bel, off):
    blk0 = off // BE
    return pl.pallas_call(
        _edge_body,
        grid=(EH // BE,),
        in_specs=[
            pl.BlockSpec((BE, H), lambda i: (i, 0)),
            pl.BlockSpec((BE, 256), lambda i: (i, 0)),
            pl.BlockSpec((BE, 128), lambda i: (i, 0)),
            pl.BlockSpec((BE, 8), lambda i: (i + blk0, 0)),
            pl.BlockSpec((H, H), lambda i: (0, 0)),
            pl.BlockSpec((1, H), lambda i: (0, 0)),
            pl.BlockSpec((1, H), lambda i: (0, 0)),
            pl.BlockSpec((1, H), lambda i: (0, 0)),
        ],
        out_specs=(
            pl.BlockSpec((BE, H), lambda i: (i, 0)),
            pl.BlockSpec((4, BE, 128), lambda i: (0, i, 0)),
        ),
        out_shape=(
            jax.ShapeDtypeStruct((EH, H), jnp.bfloat16),
            jax.ShapeDtypeStruct((4, EH, 128), jnp.float32),
        ),
    )(y, g1, g2, rp, weg, beg, gel, bel)


def _nodeupd_body(x_ref, o3_ref, sa_ref, sb_ref, gn_ref, bn_ref, out_ref):
    sa = sa_ref[...]
    sb = sb_ref[...]
    ssum = sa[0] + sa[1] + sb[0] + sb[1]
    sum_sigma = jnp.concatenate([ssum[0], ssum[1]], axis=1)
    sum_h = jnp.concatenate([ssum[2], ssum[3]], axis=1)
    h = sum_h / (sum_sigma + 1e-6)
    xu = o3_ref[...] + h
    out_ref[...] = x_ref[...] + _silu(_layernorm(xu, gn_ref[...], bn_ref[...]))


def _nodeupd(x, o3, sa, sb, gnl, bnl):
    return pl.pallas_call(
        _nodeupd_body,
        grid=(NP // BN,),
        in_specs=[
            pl.BlockSpec((BN, H), lambda i: (i, 0)),
            pl.BlockSpec((BN, H), lambda i: (i, 0)),
            pl.BlockSpec((2, 4, BN, 128), lambda i: (0, 0, i, 0)),
            pl.BlockSpec((2, 4, BN, 128), lambda i: (0, 0, i, 0)),
            pl.BlockSpec((1, H), lambda i: (0, 0)),
            pl.BlockSpec((1, H), lambda i: (0, 0)),
        ],
        out_specs=pl.BlockSpec((BN, H), lambda i: (i, 0)),
        out_shape=jax.ShapeDtypeStruct((NP, H), jnp.float32),
    )(x, o3, sa, sb, gnl, bnl)


def _readout_body(x_ref, w_ref, out_ref):
    e = jnp.dot(x_ref[...], w_ref[...], preferred_element_type=jnp.float32)
    rows = lax.broadcasted_iota(jnp.int32, (NP, 128), 0)
    e = jnp.where(rows < N, e, 0.0)
    out_ref[...] = jnp.reshape(jnp.sum(e) / N, (1, 1))


def _readout(x, wfc_pad):
    return pl.pallas_call(
        _readout_body,
        in_specs=[
            pl.BlockSpec((NP, H), lambda: (0, 0)),
            pl.BlockSpec((H, 128), lambda: (0, 0)),
        ],
        out_specs=pl.BlockSpec((1, 1), lambda: (0, 0)),
        out_shape=jax.ShapeDtypeStruct((1, 1), jnp.float32),
    )(x, wfc_pad)


def kernel(atomic_number, edge_index, r, atom_emb, Wsg, bsg, Wdg, bdg, Weg,
           beg, Wsu, bsu, Wdu, bdu, gn, bn, ge, be, Wfc, bfc):
    src = edge_index[0].astype(jnp.int32)
    dst = edge_index[1].astype(jnp.int32)
    srcp = jnp.pad(src, (0, EP - E))
    dstp = jnp.pad(dst, (0, EP - E), constant_values=PAD_DST)
    rp = jnp.pad(r, ((0, EP - E), (0, 5)))
    anp = jnp.pad(atomic_number.astype(jnp.int32), (0, NP - N))
    zeros_n = jnp.zeros((NROWS_PER_TILE, 128), jnp.float32)

    x = _gather_n256(atom_emb, anp)
    ya = _rbf(rp, 0)
    yb = _rbf(rp, EH)

    for l in range(L):
        wcat = jnp.concatenate([Wsg[l][:, _SIGINV], Wsu[l][:, _SIGINV],
                                Wdg[l][:, _SIGINV], Wdu[l]], axis=1)
        bcat = jnp.concatenate([bsg[l][_SIGINV], bsu[l][_SIGINV],
                                bdg[l][_SIGINV], bdu[l]])[None, :]
        o1, o2, o3 = _nodelin(x, wcat, bcat)
        wegl = Weg[l].astype(jnp.bfloat16)
        begl = beg[l][None, :]
        gel = ge[l][None, :]
        bel = be[l][None, :]
        g1a, g2a = _gather_edges(o1, srcp, o2, dstp, 0)
        g1b, g2b = _gather_edges(o1, srcp, o2, dstp, EH)
        ya, vals4a = _edge(ya, g1a, g2a, rp, wegl, begl, gel, bel, 0)
        sa = _scatter_k(dstp, vals4a, zeros_n, 0)
        yb, vals4b = _edge(yb, g1b, g2b, rp, wegl, begl, gel, bel, EH)
        sb = _scatter_k(dstp, vals4b, zeros_n, EH)
        x = _nodeupd(x, o3, sa, sb, gn[l][None, :], bn[l][None, :])

    wfc_pad = jnp.pad(Wfc, ((0, 0), (0, 127)))
    out = _readout(x, wfc_pad)
    return out[0, 0] + bfc[0]


# speedup vs baseline: 2.1726x; 1.0002x over previous
import functools

import numpy as np

import jax
import jax.numpy as jnp
from jax import lax
from jax.experimental import pallas as pl
from jax.experimental.pallas import tpu as pltpu
from jax.experimental.pallas import tpu_sc as plsc

N = 10000
E = 160000
H = 256
L = 3

NP = 10240
EP = 163840
NW = 32
E_PER_TILE = EP // NW
NROWS_PER_TILE = NP // 16
PAD_DST = N + 16
BE = 512
BN = 256

_SIG = np.concatenate([np.arange(128) * 2, np.arange(128) * 2 + 1])
_SIGINV = np.argsort(_SIG)

class _GatherPipe:
    def __init__(self, table_hbm, idx_hbm, out_hbm, idx_all, bufs, gsems,
                 wsems, base, npt, bg):
        pltpu.sync_copy(idx_hbm.at[pl.ds(base, npt)], idx_all)
        self.t, self.o, self.idx = table_hbm, out_hbm, idx_all
        self.bufs, self.gs, self.ws = bufs, gsems, wsems
        self.base, self.bg = base, bg
        self.nblk = npt // bg
        self.gh = [None] * self.nblk
        self.wh = [None] * self.nblk

    def step(self, b):
        if b < self.nblk:
            if b >= 2:
                self.wh[b - 2].wait()
            self.gh[b] = pltpu.async_copy(
                self.t.at[self.idx.at[pl.ds(b * self.bg, self.bg)]],
                self.bufs[b % 2], self.gs[b % 2])
        if 1 <= b <= self.nblk:
            self.gh[b - 1].wait()
            self.wh[b - 1] = pltpu.async_copy(
                self.bufs[(b - 1) % 2],
                self.o.at[pl.ds(self.base + (b - 1) * self.bg, self.bg)],
                self.ws[(b - 1) % 2])

    def drain(self):
        self.wh[self.nblk - 2].wait()
        self.wh[self.nblk - 1].wait()


def _pipelined_gather(table_hbm, idx_hbm, out_hbm, idx_all, bufs, gsems,
                      wsems, base, npt, bg):
    pipe = _GatherPipe(table_hbm, idx_hbm, out_hbm, idx_all, bufs, gsems,
                       wsems, base, npt, bg)
    for b in range(pipe.nblk + 1):
        pipe.step(b)
    pipe.drain()


@functools.lru_cache(maxsize=None)
def _make_sc_gather(n_out, n_cols, bg, dtype=jnp.float32):
    n_per_tile = n_out // NW
    nblk = n_per_tile // bg

    @functools.partial(
        pl.kernel,
        mesh=plsc.VectorSubcoreMesh(core_axis_name="c", subcore_axis_name="s"),
        out_type=jax.ShapeDtypeStruct((n_out, n_cols), dtype),
        scratch_types=[
            pltpu.VMEM((n_per_tile,), jnp.int32),
            pltpu.VMEM((bg, n_cols), dtype),
            pltpu.VMEM((bg, n_cols), dtype),
        ] + [pltpu.SemaphoreType.DMA] * 4,
    )
    def gather_k(t1, i1, o1, idx1, ba, bb, s0, s1, s2, s3):
        cid = lax.axis_index("c")
        sid = lax.axis_index("s")
        wid = sid * 2 + cid
        _pipelined_gather(t1, i1, o1, idx1, [ba, bb], [s0, s1], [s2, s3],
                          wid * n_per_tile, n_per_tile, bg)

    return gather_k


EH = EP // 2
EH_PER_TILE = EH // NW


@functools.lru_cache(maxsize=None)
def _make_sc_gather_pair(off):
    bg = 128
    nblk = EH_PER_TILE // bg

    @functools.partial(
        pl.kernel,
        mesh=plsc.VectorSubcoreMesh(core_axis_name="c", subcore_axis_name="s"),
        out_type=(
            jax.ShapeDtypeStruct((EH, 256), jnp.int32),
            jax.ShapeDtypeStruct((EH, 128), jnp.int32),
        ),
        scratch_types=[
            pltpu.VMEM((EH_PER_TILE,), jnp.int32),
            pltpu.VMEM((EH_PER_TILE,), jnp.int32),
            pltpu.VMEM((bg, 256), jnp.int32),
            pltpu.VMEM((bg, 256), jnp.int32),
            pltpu.VMEM((bg, 128), jnp.int32),
            pltpu.VMEM((bg, 128), jnp.int32),
        ] + [pltpu.SemaphoreType.DMA] * 8,
    )
    def gather_k(t1, i1, t2, i2, o1, o2, idx1, idx2, b1a, b1b, b2a, b2b,
                 s0, s1, s2, s3, s4, s5, s6, s7):
        cid = lax.axis_index("c")
        sid = lax.axis_index("s")
        wid = sid * 2 + cid
        ibase = off + wid * EH_PER_TILE
        obase = wid * EH_PER_TILE
        p1 = _GatherPipe(t1, i1, o1, idx1, [b1a, b1b], [s0, s1], [s2, s3],
                         ibase, EH_PER_TILE, bg)
        p1.base = obase
        p2 = _GatherPipe(t2, i2, o2, idx2, [b2a, b2b], [s4, s5], [s6, s7],
                         ibase, EH_PER_TILE, bg)
        p2.base = obase
        for b in range(nblk + 1):
            p1.step(b)
            p2.step(b)
        p1.drain()
        p2.drain()

    return gather_k


def _gather_edges(table1, src_idx, table2, dst_idx, off):
    t1 = lax.bitcast_convert_type(table1.reshape(NP, 256, 2), jnp.int32)
    t2 = lax.bitcast_convert_type(table2.reshape(NP, 128, 2), jnp.int32)
    return _make_sc_gather_pair(off)(t1, src_idx, t2, dst_idx)


def _gather_n256(table, idx):
    return _make_sc_gather(NP, 256, 64)(table, idx)


NBS = EH_PER_TILE // 128


@functools.lru_cache(maxsize=None)
def _make_sc_scatter(off):
    @functools.partial(
        pl.kernel,
        mesh=plsc.VectorSubcoreMesh(core_axis_name="c", subcore_axis_name="s"),
        out_type=jax.ShapeDtypeStruct((2, 4, NP, 128), jnp.float32),
        scratch_types=[
            pltpu.VMEM((NBS, 128), jnp.int32),
            pltpu.VMEM((128, 128), jnp.float32),
            pltpu.VMEM((128, 128), jnp.float32),
            pltpu.VMEM((64, 128), jnp.float32),
            pltpu.VMEM_SHARED((NP, 128), jnp.float32),
        ] + [pltpu.SemaphoreType.DMA] * 5,
    )
    def scatter_body(dst_hbm, vals_hbm, out_hbm, idx_v, va, vb, zbuf,
                     table_sh, l0, l1, a0, a1, isem):
        cid = lax.axis_index("c")
        sid = lax.axis_index("s")
        wid = sid * 2 + cid
        base = wid * EH_PER_TILE
        row0 = sid * NROWS_PER_TILE
        ih = [
            pltpu.async_copy(dst_hbm.at[pl.ds(off + base + b * 128, 128)],
                             idx_v.at[b], isem)
            for b in range(NBS)
        ]

        def zrow(i, carry):
            def zcol(j, carry2):
                zbuf[i, pl.ds(j * 16, 16)] = jnp.zeros((16,), jnp.float32)
                return carry2

            return lax.fori_loop(0, 8, zcol, carry)

        lax.fori_loop(0, 64, zrow, 0)
        for h in ih:
            h.wait()
        vbufs = [va, vb]
        lsems = [l0, l1]
        asems = [a0, a1]
        for c in range(4):
            for k in range(NROWS_PER_TILE // 64):
                pltpu.sync_copy(zbuf, table_sh.at[pl.ds(row0 + k * 64, 64)])
            plsc.subcore_barrier()
            lh = [None] * NBS
            ah = [None] * NBS
            for b in range(NBS + 1):
                if b < NBS:
                    if b >= 2:
                        ah[b - 2].wait()
                    lh[b] = pltpu.async_copy(
                        vals_hbm.at[c, pl.ds(base + b * 128, 128)],
                        vbufs[b % 2], lsems[b % 2])
                if b >= 1:
                    lh[b - 1].wait()
                    ah[b - 1] = pltpu.async_copy(
                        vbufs[(b - 1) % 2], table_sh.at[idx_v.at[b - 1]],
                        asems[(b - 1) % 2], add=True)
            ah[NBS - 2].wait()
            ah[NBS - 1].wait()
            plsc.subcore_barrier()
            pltpu.sync_copy(
                table_sh.at[pl.ds(row0, NROWS_PER_TILE)],
                out_hbm.at[cid, c, pl.ds(row0, NROWS_PER_TILE)],
            )
            plsc.subcore_barrier()

    return scatter_body


def _scatter_k(dstp, vals4, off):
    return _make_sc_scatter(off)(dstp, vals4)


def _xplor(bl):
    r_on, r_cut = 7.5, 8.0
    r2 = bl * bl
    ron2 = r_on * r_on
    rc2 = r_cut * r_cut
    smooth = ((rc2 - r2) ** 2 * (rc2 + 2.0 * r2 - 3.0 * ron2)) / (rc2 - ron2) ** 3
    return jnp.where(bl < r_on, 1.0, jnp.where(bl < r_cut, smooth, 0.0))


def _layernorm(x, g, b, eps=1e-5):
    mu = jnp.mean(x, axis=-1, keepdims=True)
    var = jnp.mean((x - mu) ** 2, axis=-1, keepdims=True)
    return (x - mu) / jnp.sqrt(var + eps) * g + b


def _silu(x):
    return x * jax.nn.sigmoid(x)


def _rbf_body(r_ref, y_ref):
    r = r_ref[...]
    bl = jnp.sqrt(jnp.sum(r * r, axis=1, keepdims=True))
    centers = lax.broadcasted_iota(jnp.int32, (1, H), 1).astype(
        jnp.float32) * (8.0 / (H - 1))
    gamma = 1.0 / (8.0 / (H - 1)) ** 2
    y_ref[...] = jnp.exp(-gamma * (bl - centers) ** 2).astype(jnp.bfloat16)


def _rbf(rp, off):
    blk0 = off // BE
    return pl.pallas_call(
        _rbf_body,
        grid=(EH // BE,),
        in_specs=[pl.BlockSpec((BE, 8), lambda i: (i + blk0, 0))],
        out_specs=pl.BlockSpec((BE, H), lambda i: (i, 0)),
        out_shape=jax.ShapeDtypeStruct((EH, H), jnp.bfloat16),
    )(rp)


def _nodelin_body(x_ref, w_ref, b_ref, o1_ref, o2_ref, o3_ref):
    acc = jnp.dot(x_ref[...], w_ref[...], preferred_element_type=jnp.float32)
    acc = acc + b_ref[...]
    o1_ref[...] = acc[:, :512].astype(jnp.bfloat16)
    o2_ref[...] = acc[:, 512:768].astype(jnp.bfloat16)
    o3_ref[...] = acc[:, 768:]


def _nodelin(x, wcat, bcat):
    return pl.pallas_call(
        _nodelin_body,
        grid=(NP // BN,),
        in_specs=[
            pl.BlockSpec((BN, H), lambda i: (i, 0)),
            pl.BlockSpec((H, 4 * H), lambda i: (0, 0)),
            pl.BlockSpec((1, 4 * H), lambda i: (0, 0)),
        ],
        out_specs=(
            pl.BlockSpec((BN, 512), lambda i: (i, 0)),
            pl.BlockSpec((BN, 256), lambda i: (i, 0)),
            pl.BlockSpec((BN, 256), lambda i: (i, 0)),
        ),
        out_shape=(
            jax.ShapeDtypeStruct((NP, 512), jnp.bfloat16),
            jax.ShapeDtypeStruct((NP, 256), jnp.bfloat16),
            jax.ShapeDtypeStruct((NP, 256), jnp.float32),
        ),
    )(x, wcat, bcat)


def _edge_body(y_ref, g1_ref, g2_ref, r_ref, weg_ref, beg_ref, ge_ref, be_ref,
               ynew_ref, vals_ref):
    y = y_ref[...]
    y32 = y.astype(jnp.float32)
    g1w = g1_ref[...]
    g2w = g2_ref[...]
    mask = jnp.int32(-65536)
    lo1 = lax.bitcast_convert_type(g1w << 16, jnp.float32)
    hi1 = lax.bitcast_convert_type(g1w & mask, jnp.float32)
    lo2 = lax.bitcast_convert_type(g2w << 16, jnp.float32)
    hi2 = lax.bitcast_convert_type(g2w & mask, jnp.float32)
    e_src = jnp.concatenate([lo1[:, :128], hi1[:, :128]], axis=1)
    bh = jnp.concatenate([lo1[:, 128:], hi1[:, 128:]], axis=1)
    e_dst = jnp.concatenate([lo2, hi2], axis=1)
    m = e_src + e_dst + beg_ref[...]
    m = m + jnp.dot(y, weg_ref[...], preferred_element_type=jnp.float32)
    r = r_ref[...]
    bl = jnp.sqrt(jnp.sum(r * r, axis=1, keepdims=True))
    cval = _xplor(bl)
    sig = jax.nn.sigmoid(m) * cval
    u = bh * sig
    vals_ref[0] = sig[:, :128]
    vals_ref[1] = sig[:, 128:]
    vals_ref[2] = u[:, :128]
    vals_ref[3] = u[:, 128:]
    yn = y32 + _silu(_layernorm(m, ge_ref[...], be_ref[...]))
    ynew_ref[...] = yn.astype(jnp.bfloat16)


def _edge(y, g1, g2, rp, weg, beg, gel, bel, off):
    blk0 = off // BE
    return pl.pallas_call(
        _edge_body,
        grid=(EH // BE,),
        in_specs=[
            pl.BlockSpec((BE, H), lambda i: (i, 0)),
            pl.BlockSpec((BE, 256), lambda i: (i, 0)),
            pl.BlockSpec((BE, 128), lambda i: (i, 0)),
            pl.BlockSpec((BE, 8), lambda i: (i + blk0, 0)),
            pl.BlockSpec((H, H), lambda i: (0, 0)),
            pl.BlockSpec((1, H), lambda i: (0, 0)),
            pl.BlockSpec((1, H), lambda i: (0, 0)),
            pl.BlockSpec((1, H), lambda i: (0, 0)),
        ],
        out_specs=(
            pl.BlockSpec((BE, H), lambda i: (i, 0)),
            pl.BlockSpec((4, BE, 128), lambda i: (0, i, 0)),
        ),
        out_shape=(
            jax.ShapeDtypeStruct((EH, H), jnp.bfloat16),
            jax.ShapeDtypeStruct((4, EH, 128), jnp.float32),
        ),
    )(y, g1, g2, rp, weg, beg, gel, bel)


def _nodeupd_body(x_ref, o3_ref, sa_ref, sb_ref, gn_ref, bn_ref, out_ref):
    sa = sa_ref[...]
    sb = sb_ref[...]
    ssum = sa[0] + sa[1] + sb[0] + sb[1]
    sum_sigma = jnp.concatenate([ssum[0], ssum[1]], axis=1)
    sum_h = jnp.concatenate([ssum[2], ssum[3]], axis=1)
    h = sum_h / (sum_sigma + 1e-6)
    xu = o3_ref[...] + h
    out_ref[...] = x_ref[...] + _silu(_layernorm(xu, gn_ref[...], bn_ref[...]))


def _nodeupd(x, o3, sa, sb, gnl, bnl):
    return pl.pallas_call(
        _nodeupd_body,
        grid=(NP // BN,),
        in_specs=[
            pl.BlockSpec((BN, H), lambda i: (i, 0)),
            pl.BlockSpec((BN, H), lambda i: (i, 0)),
            pl.BlockSpec((2, 4, BN, 128), lambda i: (0, 0, i, 0)),
            pl.BlockSpec((2, 4, BN, 128), lambda i: (0, 0, i, 0)),
            pl.BlockSpec((1, H), lambda i: (0, 0)),
            pl.BlockSpec((1, H), lambda i: (0, 0)),
        ],
        out_specs=pl.BlockSpec((BN, H), lambda i: (i, 0)),
        out_shape=jax.ShapeDtypeStruct((NP, H), jnp.float32),
    )(x, o3, sa, sb, gnl, bnl)


def _readout_body(x_ref, w_ref, out_ref):
    e = jnp.dot(x_ref[...], w_ref[...], preferred_element_type=jnp.float32)
    rows = lax.broadcasted_iota(jnp.int32, (NP, 128), 0)
    e = jnp.where(rows < N, e, 0.0)
    out_ref[...] = jnp.reshape(jnp.sum(e) / N, (1, 1))


def _readout(x, wfc_pad):
    return pl.pallas_call(
        _readout_body,
        in_specs=[
            pl.BlockSpec((NP, H), lambda: (0, 0)),
            pl.BlockSpec((H, 128), lambda: (0, 0)),
        ],
        out_specs=pl.BlockSpec((1, 1), lambda: (0, 0)),
        out_shape=jax.ShapeDtypeStruct((1, 1), jnp.float32),
    )(x, wfc_pad)


def kernel(atomic_number, edge_index, r, atom_emb, Wsg, bsg, Wdg, bdg, Weg,
           beg, Wsu, bsu, Wdu, bdu, gn, bn, ge, be, Wfc, bfc):
    src = edge_index[0].astype(jnp.int32)
    dst = edge_index[1].astype(jnp.int32)
    srcp = jnp.pad(src, (0, EP - E))
    dstp = jnp.pad(dst, (0, EP - E), constant_values=PAD_DST)
    rp = jnp.pad(r, ((0, EP - E), (0, 5)))
    anp = jnp.pad(atomic_number.astype(jnp.int32), (0, NP - N))

    x = _gather_n256(atom_emb, anp)
    ya = _rbf(rp, 0)
    yb = _rbf(rp, EH)

    for l in range(L):
        wcat = jnp.concatenate([Wsg[l][:, _SIGINV], Wsu[l][:, _SIGINV],
                                Wdg[l][:, _SIGINV], Wdu[l]], axis=1)
        bcat = jnp.concatenate([bsg[l][_SIGINV], bsu[l][_SIGINV],
                                bdg[l][_SIGINV], bdu[l]])[None, :]
        o1, o2, o3 = _nodelin(x, wcat, bcat)
        wegl = Weg[l].astype(jnp.bfloat16)
        begl = beg[l][None, :]
        gel = ge[l][None, :]
        bel = be[l][None, :]
        g1a, g2a = _gather_edges(o1, srcp, o2, dstp, 0)
        g1b, g2b = _gather_edges(o1, srcp, o2, dstp, EH)
        ya, vals4a = _edge(ya, g1a, g2a, rp, wegl, begl, gel, bel, 0)
        sa = _scatter_k(dstp, vals4a, 0)
        yb, vals4b = _edge(yb, g1b, g2b, rp, wegl, begl, gel, bel, EH)
        sb = _scatter_k(dstp, vals4b, EH)
        x = _nodeupd(x, o3, sa, sb, gn[l][None, :], bn[l][None, :])

    wfc_pad = jnp.pad(Wfc, ((0, 0), (0, 127)))
    out = _readout(x, wfc_pad)
    return out[0, 0] + bfc[0]
